# Initial kernel scaffold; baseline (speedup 1.0000x reference)
import jax
import jax.numpy as jnp
from jax.experimental import pallas as pl


def kernel(x, edge_index, batch, W1, b1, W2, b2, Wo, bo):
    raise NotImplementedError("write your pallas kernel here")


# speedup vs baseline: 10.2063x; 10.2063x over previous
import functools

import jax
import jax.numpy as jnp
from jax import lax
from jax.experimental import pallas as pl
from jax.experimental.pallas import tpu as pltpu
from jax.experimental.pallas import tpu_sc as plsc

N = 100000
E = 1600000
F_PAD = 16
H = 64
OUT = 5

NP = 102400
NC = 2
NS = 16
NW = NC * NS
B = 128
BPW = 392
EP = NW * BPW * B
ROWS_PER_TILE = NP // NS


def _sc_mesh():
    return plsc.VectorSubcoreMesh(core_axis_name="c", subcore_axis_name="s")


_SC_PARAMS = pltpu.CompilerParams(use_tc_tiling_on_sc=False)


@functools.partial(
    pl.kernel,
    out_type=jax.ShapeDtypeStruct((2 * NP,), jnp.float32),
    mesh=_sc_mesh(),
    scratch_types=[
        pltpu.VMEM((1, B), jnp.int32),
        pltpu.VMEM((B,), jnp.float32),
        pltpu.VMEM_SHARED((NP,), jnp.float32),
    ],
    compiler_params=_SC_PARAMS,
)
def _deg_kernel(dst2d, zeros1, ones1, out, idxbuf, onesbuf, acc):
    c = lax.axis_index("c")
    s = lax.axis_index("s")
    w = c * NS + s

    pltpu.sync_copy(zeros1.at[pl.ds(s * ROWS_PER_TILE, ROWS_PER_TILE)],
                    acc.at[pl.ds(s * ROWS_PER_TILE, ROWS_PER_TILE)])
    pltpu.sync_copy(ones1, onesbuf)
    plsc.subcore_barrier()

    @pl.loop(0, BPW)
    def _blk(b):
        row = w * BPW + b
        pltpu.sync_copy(dst2d.at[row], idxbuf.at[0])
        pltpu.sync_copy(onesbuf, acc.at[idxbuf.at[0]], add=True)

    plsc.subcore_barrier()
    pltpu.sync_copy(acc.at[pl.ds(s * ROWS_PER_TILE, ROWS_PER_TILE)],
                    out.at[pl.ds(c * NP + s * ROWS_PER_TILE, ROWS_PER_TILE)])


def _make_agg_kernel(G):
    @functools.partial(
        pl.kernel,
        out_type=jax.ShapeDtypeStruct((2 * G * NP, F_PAD), jnp.float32),
        mesh=_sc_mesh(),
        scratch_types=[
            pltpu.VMEM((2, B), jnp.int32),
            pltpu.VMEM((2, B), jnp.int32),
            pltpu.VMEM((2, B, F_PAD), jnp.float32),
            pltpu.SemaphoreType.DMA,
            pltpu.SemaphoreType.DMA,
            pltpu.VMEM_SHARED((NP, F_PAD), jnp.float32),
        ],
        compiler_params=_SC_PARAMS,
    )
    def _agg(src2d, dst2d, zeros2, *rest):
        srcs = rest[:G]
        out = rest[G]
        srcbuf, dstbuf, rows, sem0, sem1, acc = rest[G + 1:]
        sems = (sem0, sem1)
        c = lax.axis_index("c")
        s = lax.axis_index("s")
        w = c * NS + s

        for g in range(G):
            pltpu.sync_copy(
                zeros2.at[pl.ds(s * ROWS_PER_TILE, ROWS_PER_TILE)],
                acc.at[pl.ds(s * ROWS_PER_TILE, ROWS_PER_TILE)])
            plsc.subcore_barrier()

            @pl.loop(0, BPW // 2)
            def _blk(i):
                descs = []
                for k in range(2):
                    row = w * BPW + i * 2 + k
                    pltpu.sync_copy(src2d.at[row], srcbuf.at[k])
                    pltpu.sync_copy(dst2d.at[row], dstbuf.at[k])
                    descs.append(pltpu.async_copy(
                        srcs[g].at[srcbuf.at[k]], rows.at[k], sems[k]))
                for k in range(2):
                    descs[k].wait()
                    pltpu.sync_copy(rows.at[k], acc.at[dstbuf.at[k]],
                                    add=True)

            plsc.subcore_barrier()
            pltpu.sync_copy(
                acc.at[pl.ds(s * ROWS_PER_TILE, ROWS_PER_TILE)],
                out.at[pl.ds((c * G + g) * NP + s * ROWS_PER_TILE,
                             ROWS_PER_TILE)])
            plsc.subcore_barrier()

    return _agg


_agg1_kernel = _make_agg_kernel(1)
_agg4_kernel = _make_agg_kernel(4)


BN = 1024


def _stage1_body(deg_ref, xpad_ref, xs_ref, dinv_ref):
    d = deg_ref[0] + deg_ref[1] + 1.0
    dv = lax.rsqrt(d)
    dinv_ref[...] = dv
    xs_ref[...] = xpad_ref[...] * dv


def _stage1(deg3, xpad):
    return pl.pallas_call(
        _stage1_body,
        grid=(NP // BN,),
        in_specs=[
            pl.BlockSpec((2, BN, 1), lambda i: (0, i, 0)),
            pl.BlockSpec((BN, F_PAD), lambda i: (i, 0)),
        ],
        out_specs=[
            pl.BlockSpec((BN, F_PAD), lambda i: (i, 0)),
            pl.BlockSpec((BN, 1), lambda i: (i, 0)),
        ],
        out_shape=[
            jax.ShapeDtypeStruct((NP, F_PAD), jnp.float32),
            jax.ShapeDtypeStruct((NP, 1), jnp.float32),
        ],
    )(deg3, xpad)


def _stage2_body(a1_ref, xs_ref, dinv_ref, w1_ref, b1_ref, *out_refs):
    t = (a1_ref[0] + a1_ref[1] + xs_ref[...]) * dinv_ref[...]
    h = jnp.dot(t, w1_ref[...], preferred_element_type=jnp.float32)
    h = h + b1_ref[...]
    r = jnp.maximum(h, 0.0) * dinv_ref[...]
    for g in range(4):
        out_refs[g][...] = r[:, g * F_PAD:(g + 1) * F_PAD]


def _stage2(a1, xs, dinv, w1p, b1r):
    return pl.pallas_call(
        _stage2_body,
        grid=(NP // BN,),
        in_specs=[
            pl.BlockSpec((2, BN, F_PAD), lambda i: (0, i, 0)),
            pl.BlockSpec((BN, F_PAD), lambda i: (i, 0)),
            pl.BlockSpec((BN, 1), lambda i: (i, 0)),
            pl.BlockSpec((F_PAD, H), lambda i: (0, 0)),
            pl.BlockSpec((1, H), lambda i: (0, 0)),
        ],
        out_specs=[pl.BlockSpec((BN, F_PAD), lambda i: (i, 0))] * 4,
        out_shape=[jax.ShapeDtypeStruct((NP, F_PAD), jnp.float32)] * 4,
    )(a1, xs, dinv, w1p, b1r)


def _stage3_body(a2_ref, h0_ref, h1_ref, h2_ref, h3_ref, dinv_ref,
                 w2_ref, b2_ref, wo_ref, bo_ref, out_ref):
    hs = jnp.concatenate(
        [h0_ref[...], h1_ref[...], h2_ref[...], h3_ref[...]], axis=1)
    agg = jnp.concatenate(
        [a2_ref[g] + a2_ref[4 + g] for g in range(4)], axis=1)
    t2 = (agg + hs) * dinv_ref[...]
    h2 = jnp.dot(t2, w2_ref[...], preferred_element_type=jnp.float32)
    h2 = h2 + b2_ref[...]
    r2 = jnp.maximum(h2, 0.0)
    lg = jnp.dot(r2, wo_ref[...], preferred_element_type=jnp.float32)
    lg = lg + bo_ref[...]
    m = jnp.max(lg, axis=1, keepdims=True)
    e = jnp.exp(lg - m)
    out_ref[...] = e / jnp.sum(e, axis=1, keepdims=True)


def _stage3(a2, hs4, dinv, w2, b2r, wop, bo8):
    return pl.pallas_call(
        _stage3_body,
        grid=(NP // BN,),
        in_specs=[
            pl.BlockSpec((8, BN, F_PAD), lambda i: (0, i, 0)),
            pl.BlockSpec((BN, F_PAD), lambda i: (i, 0)),
            pl.BlockSpec((BN, F_PAD), lambda i: (i, 0)),
            pl.BlockSpec((BN, F_PAD), lambda i: (i, 0)),
            pl.BlockSpec((BN, F_PAD), lambda i: (i, 0)),
            pl.BlockSpec((BN, 1), lambda i: (i, 0)),
            pl.BlockSpec((H, H), lambda i: (0, 0)),
            pl.BlockSpec((1, H), lambda i: (0, 0)),
            pl.BlockSpec((H, 8), lambda i: (0, 0)),
            pl.BlockSpec((1, 8), lambda i: (0, 0)),
        ],
        out_specs=pl.BlockSpec((BN, 8), lambda i: (i, 0)),
        out_shape=jax.ShapeDtypeStruct((NP, 8), jnp.float32),
    )(a2, *hs4, dinv, w2, b2r, wop, bo8)


def kernel(x, edge_index, batch, W1, b1, W2, b2, Wo, bo):
    del batch
    f_in = x.shape[2]

    x_last = x[:, -1, :]
    xpad = jnp.zeros((NP, F_PAD), jnp.float32).at[:N, :f_in].set(x_last)

    pad_idx = (N + (jnp.arange(EP - E, dtype=jnp.int32) % 64))
    src = jnp.concatenate([edge_index[0], pad_idx]).reshape(NW * BPW, B)
    dst = jnp.concatenate([edge_index[1], pad_idx]).reshape(NW * BPW, B)

    zeros1 = jnp.zeros((NP,), jnp.float32)
    zeros2 = jnp.zeros((NP, F_PAD), jnp.float32)
    ones1 = jnp.ones((B,), jnp.float32)

    w1p = jnp.zeros((F_PAD, H), jnp.float32).at[:f_in, :].set(W1)
    b1r = b1.reshape(1, H)
    b2r = b2.reshape(1, H)
    wop = jnp.zeros((H, 8), jnp.float32).at[:, :OUT].set(Wo)
    bo8 = jnp.full((1, 8), -1e30, jnp.float32).at[0, :OUT].set(bo)

    deg2 = _deg_kernel(dst, zeros1, ones1)
    deg3 = deg2.reshape(2, NP, 1)

    xs, dinv = _stage1(deg3, xpad)

    a1 = _agg1_kernel(src, dst, zeros2, xs).reshape(2, NP, F_PAD)

    hs4 = _stage2(a1, xs, dinv, w1p, b1r)

    a2 = _agg4_kernel(src, dst, zeros2, *hs4).reshape(8, NP, F_PAD)

    probs = _stage3(a2, hs4, dinv, W2, b2r, wop, bo8)

    return probs[:N, :OUT]


# speedup vs baseline: 20.3668x; 1.9955x over previous
import functools

import jax
import jax.numpy as jnp
from jax import lax
from jax.experimental import pallas as pl
from jax.experimental.pallas import tpu as pltpu
from jax.experimental.pallas import tpu_sc as plsc

N = 100000
E = 1600000
F_PAD = 16
H = 64
OUT = 5

NP = 102400
NC = 2
NS = 16
NW = NC * NS
SB = 512
SBB = SB // 128
SBPW = 98
EPW = SBPW * SB
EP = NW * EPW
ROWS_PER_TILE = NP // NS


def _sc_mesh():
    return plsc.VectorSubcoreMesh(core_axis_name="c", subcore_axis_name="s")


_SC_PARAMS = pltpu.CompilerParams(use_tc_tiling_on_sc=False)


@functools.partial(
    pl.kernel,
    out_type=jax.ShapeDtypeStruct((2 * NP,), jnp.float32),
    mesh=_sc_mesh(),
    scratch_types=[
        pltpu.VMEM((2, SBB, 128), jnp.int32),
        pltpu.VMEM((128,), jnp.float32),
        pltpu.SemaphoreType.DMA,
        pltpu.SemaphoreType.DMA,
        pltpu.VMEM_SHARED((NP,), jnp.float32),
    ],
    compiler_params=_SC_PARAMS,
)
def _deg_kernel(dst2d, zeros1, ones1, out, dstbuf, onesbuf, sem0, sem1, acc):
    c = lax.axis_index("c")
    s = lax.axis_index("s")
    w = c * NS + s
    sems = (sem0, sem1)

    pltpu.sync_copy(zeros1.at[pl.ds(s * ROWS_PER_TILE, ROWS_PER_TILE)],
                    acc.at[pl.ds(s * ROWS_PER_TILE, ROWS_PER_TILE)])
    pltpu.sync_copy(ones1, onesbuf)
    plsc.subcore_barrier()

    @pl.loop(0, SBPW // 2)
    def _pair(t):
        for k in range(2):
            sb = t * 2 + k

            @pl.when(t >= 1)
            def _drain():
                pltpu.make_async_copy(dst2d.at[pl.ds(0, SBB)],
                                      dstbuf.at[k], sems[k]).wait()

            row0 = w * (SBPW * SBB) + sb * SBB
            pltpu.sync_copy(dst2d.at[pl.ds(row0, SBB)], dstbuf.at[k])
            for j in range(SBB):
                pltpu.async_copy(onesbuf, acc.at[dstbuf.at[k, j]], sems[k],
                                 add=True)

    for k in range(2):
        pltpu.make_async_copy(dst2d.at[pl.ds(0, SBB)], dstbuf.at[k],
                              sems[k]).wait()

    plsc.subcore_barrier()
    pltpu.sync_copy(acc.at[pl.ds(s * ROWS_PER_TILE, ROWS_PER_TILE)],
                    out.at[pl.ds(c * NP + s * ROWS_PER_TILE, ROWS_PER_TILE)])


def _make_agg_kernel(G):
    @functools.partial(
        pl.kernel,
        out_type=jax.ShapeDtypeStruct((2 * G * NP, F_PAD), jnp.float32),
        mesh=_sc_mesh(),
        scratch_types=[
            pltpu.VMEM((2, SB), jnp.int32),
            pltpu.VMEM((2, SBB, 128), jnp.int32),
            pltpu.VMEM((2, SB, F_PAD), jnp.float32),
            pltpu.SemaphoreType.DMA,
            pltpu.SemaphoreType.DMA,
            pltpu.SemaphoreType.DMA,
            pltpu.SemaphoreType.DMA,
            pltpu.VMEM_SHARED((NP, F_PAD), jnp.float32),
        ],
        compiler_params=_SC_PARAMS,
    )
    def _agg(src1d, dst2d, zeros2, *rest):
        srcs = rest[:G]
        out = rest[G]
        srcbuf, dstbuf, rows, sg0, sg1, ss0, ss1, acc = rest[G + 1:]
        sem_g = (sg0, sg1)
        sem_s = (ss0, ss1)
        c = lax.axis_index("c")
        s = lax.axis_index("s")
        w = c * NS + s

        def start_scatters(g, o):
            for j in range(SBB):
                pltpu.async_copy(rows.at[o, pl.ds(j * 128, 128)],
                                 acc.at[dstbuf.at[o, j]], sem_s[o], add=True)

        def drain_gather(g, o):
            pltpu.make_async_copy(srcs[g].at[pl.ds(0, SB)], rows.at[o],
                                  sem_g[o]).wait()

        def drain_scatter(g, k):
            pltpu.make_async_copy(srcs[g].at[pl.ds(0, SB)], rows.at[k],
                                  sem_s[k]).wait()

        for g in range(G):
            pltpu.sync_copy(
                zeros2.at[pl.ds(s * ROWS_PER_TILE, ROWS_PER_TILE)],
                acc.at[pl.ds(s * ROWS_PER_TILE, ROWS_PER_TILE)])
            plsc.subcore_barrier()

            @pl.loop(0, SBPW // 2)
            def _pair(t):
                for k in range(2):
                    sb = t * 2 + k
                    o = 1 - k

                    @pl.when(t >= 1)
                    def _drain_s():
                        drain_scatter(g, k)

                    e0 = w * EPW + sb * SB
                    row0 = w * (SBPW * SBB) + sb * SBB
                    pltpu.sync_copy(src1d.at[pl.ds(e0, SB)], srcbuf.at[k])
                    pltpu.sync_copy(dst2d.at[pl.ds(row0, SBB)], dstbuf.at[k])
                    pltpu.async_copy(srcs[g].at[srcbuf.at[k]], rows.at[k],
                                     sem_g[k])

                    if k == 0:
                        @pl.when(t >= 1)
                        def _pipe():
                            drain_gather(g, o)
                            start_scatters(g, o)
                    else:
                        drain_gather(g, o)
                        start_scatters(g, o)

            drain_gather(g, 1)
            start_scatters(g, 1)
            drain_scatter(g, 0)
            drain_scatter(g, 1)

            plsc.subcore_barrier()
            pltpu.sync_copy(
                acc.at[pl.ds(s * ROWS_PER_TILE, ROWS_PER_TILE)],
                out.at[pl.ds((c * G + g) * NP + s * ROWS_PER_TILE,
                             ROWS_PER_TILE)])
            plsc.subcore_barrier()

    return _agg


_agg1_kernel = _make_agg_kernel(1)
_agg4_kernel = _make_agg_kernel(4)


BN = 1024


def _stage1_body(deg_ref, xpad_ref, xs_ref, dinv_ref):
    d = deg_ref[0] + deg_ref[1] + 1.0
    dv = lax.rsqrt(d)
    dinv_ref[...] = dv
    xs_ref[...] = xpad_ref[...] * dv


def _stage1(deg3, xpad):
    return pl.pallas_call(
        _stage1_body,
        grid=(NP // BN,),
        in_specs=[
            pl.BlockSpec((2, BN, 1), lambda i: (0, i, 0)),
            pl.BlockSpec((BN, F_PAD), lambda i: (i, 0)),
        ],
        out_specs=[
            pl.BlockSpec((BN, F_PAD), lambda i: (i, 0)),
            pl.BlockSpec((BN, 1), lambda i: (i, 0)),
        ],
        out_shape=[
            jax.ShapeDtypeStruct((NP, F_PAD), jnp.float32),
            jax.ShapeDtypeStruct((NP, 1), jnp.float32),
        ],
    )(deg3, xpad)


def _stage2_body(a1_ref, xs_ref, dinv_ref, w1_ref, b1_ref, *out_refs):
    t = (a1_ref[0] + a1_ref[1] + xs_ref[...]) * dinv_ref[...]
    h = jnp.dot(t, w1_ref[...], preferred_element_type=jnp.float32)
    h = h + b1_ref[...]
    r = jnp.maximum(h, 0.0) * dinv_ref[...]
    for g in range(4):
        out_refs[g][...] = r[:, g * F_PAD:(g + 1) * F_PAD]


def _stage2(a1, xs, dinv, w1p, b1r):
    return pl.pallas_call(
        _stage2_body,
        grid=(NP // BN,),
        in_specs=[
            pl.BlockSpec((2, BN, F_PAD), lambda i: (0, i, 0)),
            pl.BlockSpec((BN, F_PAD), lambda i: (i, 0)),
            pl.BlockSpec((BN, 1), lambda i: (i, 0)),
            pl.BlockSpec((F_PAD, H), lambda i: (0, 0)),
            pl.BlockSpec((1, H), lambda i: (0, 0)),
        ],
        out_specs=[pl.BlockSpec((BN, F_PAD), lambda i: (i, 0))] * 4,
        out_shape=[jax.ShapeDtypeStruct((NP, F_PAD), jnp.float32)] * 4,
    )(a1, xs, dinv, w1p, b1r)


def _stage3_body(a2_ref, h0_ref, h1_ref, h2_ref, h3_ref, dinv_ref,
                 w2_ref, b2_ref, wo_ref, bo_ref, out_ref):
    hs = jnp.concatenate(
        [h0_ref[...], h1_ref[...], h2_ref[...], h3_ref[...]], axis=1)
    agg = jnp.concatenate(
        [a2_ref[g] + a2_ref[4 + g] for g in range(4)], axis=1)
    t2 = (agg + hs) * dinv_ref[...]
    h2 = jnp.dot(t2, w2_ref[...], preferred_element_type=jnp.float32)
    h2 = h2 + b2_ref[...]
    r2 = jnp.maximum(h2, 0.0)
    lg = jnp.dot(r2, wo_ref[...], preferred_element_type=jnp.float32)
    lg = lg + bo_ref[...]
    m = jnp.max(lg, axis=1, keepdims=True)
    e = jnp.exp(lg - m)
    out_ref[...] = e / jnp.sum(e, axis=1, keepdims=True)


def _stage3(a2, hs4, dinv, w2, b2r, wop, bo8):
    return pl.pallas_call(
        _stage3_body,
        grid=(NP // BN,),
        in_specs=[
            pl.BlockSpec((8, BN, F_PAD), lambda i: (0, i, 0)),
            pl.BlockSpec((BN, F_PAD), lambda i: (i, 0)),
            pl.BlockSpec((BN, F_PAD), lambda i: (i, 0)),
            pl.BlockSpec((BN, F_PAD), lambda i: (i, 0)),
            pl.BlockSpec((BN, F_PAD), lambda i: (i, 0)),
            pl.BlockSpec((BN, 1), lambda i: (i, 0)),
            pl.BlockSpec((H, H), lambda i: (0, 0)),
            pl.BlockSpec((1, H), lambda i: (0, 0)),
            pl.BlockSpec((H, 8), lambda i: (0, 0)),
            pl.BlockSpec((1, 8), lambda i: (0, 0)),
        ],
        out_specs=pl.BlockSpec((BN, 8), lambda i: (i, 0)),
        out_shape=jax.ShapeDtypeStruct((NP, 8), jnp.float32),
    )(a2, *hs4, dinv, w2, b2r, wop, bo8)


def kernel(x, edge_index, batch, W1, b1, W2, b2, Wo, bo):
    del batch
    f_in = x.shape[2]

    x_last = x[:, -1, :]
    xpad = jnp.zeros((NP, F_PAD), jnp.float32).at[:N, :f_in].set(x_last)

    pad_idx = (N + (jnp.arange(EP - E, dtype=jnp.int32) % (NP - N)))
    src = jnp.concatenate([edge_index[0], pad_idx])
    dst = jnp.concatenate([edge_index[1], pad_idx]).reshape(EP // 128, 128)

    zeros1 = jnp.zeros((NP,), jnp.float32)
    zeros2 = jnp.zeros((NP, F_PAD), jnp.float32)
    ones1 = jnp.ones((128,), jnp.float32)

    w1p = jnp.zeros((F_PAD, H), jnp.float32).at[:f_in, :].set(W1)
    b1r = b1.reshape(1, H)
    b2r = b2.reshape(1, H)
    wop = jnp.zeros((H, 8), jnp.float32).at[:, :OUT].set(Wo)
    bo8 = jnp.full((1, 8), -1e30, jnp.float32).at[0, :OUT].set(bo)

    deg2 = _deg_kernel(dst, zeros1, ones1)
    deg3 = deg2.reshape(2, NP, 1)

    xs, dinv = _stage1(deg3, xpad)

    a1 = _agg1_kernel(src, dst, zeros2, xs).reshape(2, NP, F_PAD)

    hs4 = _stage2(a1, xs, dinv, w1p, b1r)

    a2 = _agg4_kernel(src, dst, zeros2, *hs4).reshape(8, NP, F_PAD)

    probs = _stage3(a2, hs4, dinv, W2, b2r, wop, bo8)

    return probs[:N, :OUT]


# speedup vs baseline: 37.4622x; 1.8394x over previous
import functools

import jax
import jax.numpy as jnp
from jax import lax
from jax.experimental import pallas as pl
from jax.experimental.pallas import tpu as pltpu
from jax.experimental.pallas import tpu_sc as plsc

N = 100000
E = 1600000
F_PAD = 16
H = 64
OUT = 5

NP = 102400
NC = 2
NS = 16
NW = NC * NS
SB = 512
SBB = SB // 128
SBPW = 98
EPW = SBPW * SB
EP = NW * EPW
ROWS_PER_TILE = NP // NS


def _sc_mesh():
    return plsc.VectorSubcoreMesh(core_axis_name="c", subcore_axis_name="s")


_SC_PARAMS = pltpu.CompilerParams(use_tc_tiling_on_sc=False)


DEG_CHUNK = 800


@functools.partial(
    pl.kernel,
    out_type=jax.ShapeDtypeStruct((2 * NP * F_PAD,), jnp.float32),
    mesh=_sc_mesh(),
    scratch_types=[
        pltpu.VMEM((2, SBB, 128), jnp.int32),
        pltpu.VMEM((128,), jnp.float32),
        pltpu.VMEM((ROWS_PER_TILE,), jnp.float32),
        pltpu.VMEM((DEG_CHUNK * F_PAD,), jnp.float32),
        pltpu.SemaphoreType.DMA,
        pltpu.SemaphoreType.DMA,
        pltpu.VMEM_SHARED((NP,), jnp.float32),
    ],
    compiler_params=_SC_PARAMS,
)
def _deg_kernel(dst2d, zeros1, ones1, out, dstbuf, onesbuf, degv, bc,
                sem0, sem1, acc):
    c = lax.axis_index("c")
    s = lax.axis_index("s")
    w = c * NS + s
    sems = (sem0, sem1)

    pltpu.sync_copy(zeros1.at[pl.ds(s * ROWS_PER_TILE, ROWS_PER_TILE)],
                    acc.at[pl.ds(s * ROWS_PER_TILE, ROWS_PER_TILE)])
    pltpu.sync_copy(ones1, onesbuf)
    plsc.subcore_barrier()

    @pl.loop(0, SBPW // 2)
    def _pair(t):
        for k in range(2):
            sb = t * 2 + k

            @pl.when(t >= 1)
            def _drain():
                pltpu.make_async_copy(dst2d.at[pl.ds(0, SBB)],
                                      dstbuf.at[k], sems[k]).wait()

            row0 = w * (SBPW * SBB) + sb * SBB
            pltpu.sync_copy(dst2d.at[pl.ds(row0, SBB)], dstbuf.at[k])
            for j in range(SBB):
                pltpu.async_copy(onesbuf, acc.at[dstbuf.at[k, j]], sems[k],
                                 add=True)

    for k in range(2):
        pltpu.make_async_copy(dst2d.at[pl.ds(0, SBB)], dstbuf.at[k],
                              sems[k]).wait()

    plsc.subcore_barrier()
    pltpu.sync_copy(acc.at[pl.ds(s * ROWS_PER_TILE, ROWS_PER_TILE)], degv)
    for chunk in range(ROWS_PER_TILE // DEG_CHUNK):
        @pl.loop(0, DEG_CHUNK // 16)
        def _bcast(i):
            v = degv[pl.ds(chunk * DEG_CHUNK + i * 16, 16)]
            for j in range(16):
                bc[pl.ds((i * 16 + j) * F_PAD, F_PAD)] = (
                    jnp.broadcast_to(v[j], (F_PAD,)))

        pltpu.sync_copy(
            bc,
            out.at[pl.ds((c * NP + s * ROWS_PER_TILE + chunk * DEG_CHUNK)
                         * F_PAD, DEG_CHUNK * F_PAD)])


def _make_agg_kernel(G):
    @functools.partial(
        pl.kernel,
        out_type=jax.ShapeDtypeStruct((2 * G * NP, F_PAD), jnp.float32),
        mesh=_sc_mesh(),
        scratch_types=[
            pltpu.VMEM((2, SB), jnp.int32),
            pltpu.VMEM((2, SBB, 128), jnp.int32),
            pltpu.VMEM((2, SB, F_PAD), jnp.float32),
            pltpu.SemaphoreType.DMA,
            pltpu.SemaphoreType.DMA,
            pltpu.SemaphoreType.DMA,
            pltpu.SemaphoreType.DMA,
            pltpu.VMEM_SHARED((NP, F_PAD), jnp.float32),
        ],
        compiler_params=_SC_PARAMS,
    )
    def _agg(src1d, dst2d, zeros2, *rest):
        srcs = rest[:G]
        out = rest[G]
        srcbuf, dstbuf, rows, sg0, sg1, ss0, ss1, acc = rest[G + 1:]
        sem_g = (sg0, sg1)
        sem_s = (ss0, ss1)
        c = lax.axis_index("c")
        s = lax.axis_index("s")
        w = c * NS + s

        def start_scatters(g, o):
            for j in range(SBB):
                pltpu.async_copy(rows.at[o, pl.ds(j * 128, 128)],
                                 acc.at[dstbuf.at[o, j]], sem_s[o], add=True)

        def drain_gather(g, o):
            pltpu.make_async_copy(srcs[g].at[pl.ds(0, SB)], rows.at[o],
                                  sem_g[o]).wait()

        def drain_scatter(g, k):
            pltpu.make_async_copy(srcs[g].at[pl.ds(0, SB)], rows.at[k],
                                  sem_s[k]).wait()

        for g in range(G):
            pltpu.sync_copy(
                zeros2.at[pl.ds(s * ROWS_PER_TILE, ROWS_PER_TILE)],
                acc.at[pl.ds(s * ROWS_PER_TILE, ROWS_PER_TILE)])
            plsc.subcore_barrier()

            @pl.loop(0, SBPW // 2)
            def _pair(t):
                for k in range(2):
                    sb = t * 2 + k
                    o = 1 - k

                    @pl.when(t >= 1)
                    def _drain_s():
                        drain_scatter(g, k)

                    e0 = w * EPW + sb * SB
                    row0 = w * (SBPW * SBB) + sb * SBB
                    pltpu.sync_copy(src1d.at[pl.ds(e0, SB)], srcbuf.at[k])
                    pltpu.sync_copy(dst2d.at[pl.ds(row0, SBB)], dstbuf.at[k])
                    pltpu.async_copy(srcs[g].at[srcbuf.at[k]], rows.at[k],
                                     sem_g[k])

                    if k == 0:
                        @pl.when(t >= 1)
                        def _pipe():
                            drain_gather(g, o)
                            start_scatters(g, o)
                    else:
                        drain_gather(g, o)
                        start_scatters(g, o)

            drain_gather(g, 1)
            start_scatters(g, 1)
            drain_scatter(g, 0)
            drain_scatter(g, 1)

            plsc.subcore_barrier()
            pltpu.sync_copy(
                acc.at[pl.ds(s * ROWS_PER_TILE, ROWS_PER_TILE)],
                out.at[pl.ds((c * G + g) * NP + s * ROWS_PER_TILE,
                             ROWS_PER_TILE)])
            plsc.subcore_barrier()

    return _agg


_agg1_kernel = _make_agg_kernel(1)
_agg4_kernel = _make_agg_kernel(4)


R = NP // 8
BB = 1600
GRID = R // BB


def _stage1_body(deg_ref, xpad_ref, xs_ref, dinv_ref):
    d = deg_ref[0] + deg_ref[1] + 1.0
    dv = lax.rsqrt(d)
    dinv_ref[...] = dv
    xs_ref[...] = xpad_ref[...] * dv


def _stage1(deg16p, xpadp):
    return pl.pallas_call(
        _stage1_body,
        grid=(GRID,),
        in_specs=[
            pl.BlockSpec((2, BB, 128), lambda i: (0, i, 0)),
            pl.BlockSpec((BB, 128), lambda i: (i, 0)),
        ],
        out_specs=[
            pl.BlockSpec((BB, 128), lambda i: (i, 0)),
            pl.BlockSpec((BB, 128), lambda i: (i, 0)),
        ],
        out_shape=[
            jax.ShapeDtypeStruct((R, 128), jnp.float32),
            jax.ShapeDtypeStruct((R, 128), jnp.float32),
        ],
    )(deg16p, xpadp)


def _stage2_body(a1_ref, xs_ref, dinv_ref, w1s_ref, b1c_ref, *out_refs):
    t = (a1_ref[0] + a1_ref[1] + xs_ref[...]) * dinv_ref[...]
    h = jnp.dot(t, w1s_ref[...], preferred_element_type=jnp.float32)
    h = h + b1c_ref[...]
    r = jnp.maximum(h, 0.0)
    dv = dinv_ref[...]
    for g in range(4):
        out_refs[g][...] = r[:, g * 128:(g + 1) * 128] * dv


def _stage2(a1p, xsp, dinvp, w1s, b1c):
    return pl.pallas_call(
        _stage2_body,
        grid=(GRID,),
        in_specs=[
            pl.BlockSpec((2, BB, 128), lambda i: (0, i, 0)),
            pl.BlockSpec((BB, 128), lambda i: (i, 0)),
            pl.BlockSpec((BB, 128), lambda i: (i, 0)),
            pl.BlockSpec((128, 512), lambda i: (0, 0)),
            pl.BlockSpec((1, 512), lambda i: (0, 0)),
        ],
        out_specs=[pl.BlockSpec((BB, 128), lambda i: (i, 0))] * 4,
        out_shape=[jax.ShapeDtypeStruct((R, 128), jnp.float32)] * 4,
    )(a1p, xsp, dinvp, w1s, b1c)


def _stage3_body(a2_ref, h0_ref, h1_ref, h2_ref, h3_ref, dinv_ref,
                 w2s_ref, b2c_ref, wos_ref, boc_ref, sumg_ref, out_ref):
    hs = (h0_ref[...], h1_ref[...], h2_ref[...], h3_ref[...])
    dv = dinv_ref[...]
    t2 = jnp.concatenate(
        [(a2_ref[g] + a2_ref[4 + g] + hs[g]) * dv for g in range(4)], axis=1)
    h2 = jnp.dot(t2, w2s_ref[...], preferred_element_type=jnp.float32)
    h2 = h2 + b2c_ref[...]
    r2 = jnp.maximum(h2, 0.0)
    lg = jnp.dot(r2, wos_ref[...], preferred_element_type=jnp.float32)
    lg = lg + boc_ref[...]
    m = jnp.max(lg, axis=1, keepdims=True)
    e = jnp.exp(lg - m)
    ssum = jnp.dot(e, sumg_ref[...], preferred_element_type=jnp.float32)
    out_ref[...] = e / ssum


def _stage3(a2p, hs4, dinvp, w2s, b2c, wos, boc, sumg):
    return pl.pallas_call(
        _stage3_body,
        grid=(GRID,),
        in_specs=[
            pl.BlockSpec((8, BB, 128), lambda i: (0, i, 0)),
            pl.BlockSpec((BB, 128), lambda i: (i, 0)),
            pl.BlockSpec((BB, 128), lambda i: (i, 0)),
            pl.BlockSpec((BB, 128), lambda i: (i, 0)),
            pl.BlockSpec((BB, 128), lambda i: (i, 0)),
            pl.BlockSpec((BB, 128), lambda i: (i, 0)),
            pl.BlockSpec((512, 512), lambda i: (0, 0)),
            pl.BlockSpec((1, 512), lambda i: (0, 0)),
            pl.BlockSpec((512, 64), lambda i: (0, 0)),
            pl.BlockSpec((1, 64), lambda i: (0, 0)),
            pl.BlockSpec((64, 64), lambda i: (0, 0)),
        ],
        out_specs=pl.BlockSpec((BB, 64), lambda i: (i, 0)),
        out_shape=jax.ShapeDtypeStruct((R, 64), jnp.float32),
    )(a2p, *hs4, dinvp, w2s, b2c, wos, boc, sumg)


def kernel(x, edge_index, batch, W1, b1, W2, b2, Wo, bo):
    del batch
    f_in = x.shape[2]
    eye8 = jnp.eye(8, dtype=jnp.float32)

    x_last = x[:, -1, :]
    xpad = jnp.zeros((NP, F_PAD), jnp.float32).at[:N, :f_in].set(x_last)
    xpadp = xpad.reshape(R, 128)

    pad_idx = (N + (jnp.arange(EP - E, dtype=jnp.int32) % (NP - N)))
    src = jnp.concatenate([edge_index[0], pad_idx])
    dst = jnp.concatenate([edge_index[1], pad_idx]).reshape(EP // 128, 128)

    zeros1 = jnp.zeros((NP,), jnp.float32)
    zeros2 = jnp.zeros((NP, F_PAD), jnp.float32)
    ones1 = jnp.ones((128,), jnp.float32)

    w1p = jnp.zeros((F_PAD, H), jnp.float32).at[:f_in, :].set(W1)
    w1s = jnp.concatenate(
        [jnp.kron(eye8, w1p[:, g * 16:(g + 1) * 16]) for g in range(4)],
        axis=1)
    b1c = jnp.tile(b1.reshape(4, 16), (1, 8)).reshape(1, 512)
    w2r = W2.reshape(4, 16, 4, 16)
    w2s = jnp.concatenate(
        [jnp.concatenate([jnp.kron(eye8, w2r[gi, :, go, :])
                          for gi in range(4)], axis=0)
         for go in range(4)], axis=1)
    b2c = jnp.tile(b2.reshape(4, 16), (1, 8)).reshape(1, 512)
    wop = jnp.zeros((H, 8), jnp.float32).at[:, :OUT].set(Wo)
    wos = jnp.concatenate(
        [jnp.kron(eye8, wop[g * 16:(g + 1) * 16, :]) for g in range(4)],
        axis=0)
    bo8 = jnp.full((8,), -1e30, jnp.float32).at[:OUT].set(bo)
    boc = jnp.tile(bo8, 8).reshape(1, 64)
    sumg = jnp.kron(eye8, jnp.ones((8, 8), jnp.float32))

    deg16p = _deg_kernel(dst, zeros1, ones1).reshape(2, R, 128)

    xsp, dinvp = _stage1(deg16p, xpadp)

    a1 = _agg1_kernel(src, dst, zeros2, xsp.reshape(NP, F_PAD))
    a1p = a1.reshape(2, R, 128)

    hs4 = _stage2(a1p, xsp, dinvp, w1s, b1c)

    a2 = _agg4_kernel(src, dst, zeros2,
                      *[h.reshape(NP, F_PAD) for h in hs4])
    a2p = a2.reshape(8, R, 128)

    probs = _stage3(a2p, hs4, dinvp, w2s, b2c, wos, boc, sumg)

    return probs.reshape(NP, 8)[:N, :OUT]


# speedup vs baseline: 37.4872x; 1.0007x over previous
import functools

import jax
import jax.numpy as jnp
from jax import lax
from jax.experimental import pallas as pl
from jax.experimental.pallas import tpu as pltpu
from jax.experimental.pallas import tpu_sc as plsc

N = 100000
E = 1600000
F_PAD = 16
H = 64
OUT = 5

NP = 102400
NC = 2
NS = 16
NW = NC * NS
SB = 512
SBB = SB // 128
SBPW = 98
EPW = SBPW * SB
EP = NW * EPW
ROWS_PER_TILE = NP // NS


def _sc_mesh():
    return plsc.VectorSubcoreMesh(core_axis_name="c", subcore_axis_name="s")


_SC_PARAMS = pltpu.CompilerParams(use_tc_tiling_on_sc=False)


DEG_CHUNK = 800


@functools.partial(
    pl.kernel,
    out_type=jax.ShapeDtypeStruct((2 * NP * F_PAD,), jnp.float32),
    mesh=_sc_mesh(),
    scratch_types=[
        pltpu.VMEM((2, SB), jnp.int32),
        pltpu.VMEM((SB,), jnp.float32),
        pltpu.VMEM((ROWS_PER_TILE,), jnp.float32),
        pltpu.VMEM((DEG_CHUNK * F_PAD,), jnp.float32),
        pltpu.SemaphoreType.DMA,
        pltpu.SemaphoreType.DMA,
        pltpu.VMEM_SHARED((NP,), jnp.float32),
    ],
    compiler_params=_SC_PARAMS,
)
def _deg_kernel(dst1d, zeros1, ones1, out, dstbuf, onesbuf, degv, bc,
                sem0, sem1, acc):
    c = lax.axis_index("c")
    s = lax.axis_index("s")
    w = c * NS + s
    sems = (sem0, sem1)

    pltpu.sync_copy(zeros1.at[pl.ds(s * ROWS_PER_TILE, ROWS_PER_TILE)],
                    acc.at[pl.ds(s * ROWS_PER_TILE, ROWS_PER_TILE)])
    pltpu.sync_copy(ones1, onesbuf)
    plsc.subcore_barrier()

    @pl.loop(0, SBPW // 2)
    def _pair(t):
        for k in range(2):
            sb = t * 2 + k

            @pl.when(t >= 1)
            def _drain():
                pltpu.make_async_copy(dst1d.at[pl.ds(0, SB)],
                                      dstbuf.at[k], sems[k]).wait()

            e0 = w * EPW + sb * SB
            pltpu.sync_copy(dst1d.at[pl.ds(e0, SB)], dstbuf.at[k])
            pltpu.async_copy(onesbuf, acc.at[dstbuf.at[k]], sems[k],
                             add=True)

    for k in range(2):
        pltpu.make_async_copy(dst1d.at[pl.ds(0, SB)], dstbuf.at[k],
                              sems[k]).wait()

    plsc.subcore_barrier()
    pltpu.sync_copy(acc.at[pl.ds(s * ROWS_PER_TILE, ROWS_PER_TILE)], degv)
    for chunk in range(ROWS_PER_TILE // DEG_CHUNK):
        @pl.loop(0, DEG_CHUNK // 16)
        def _bcast(i):
            v = degv[pl.ds(chunk * DEG_CHUNK + i * 16, 16)]
            for j in range(16):
                bc[pl.ds((i * 16 + j) * F_PAD, F_PAD)] = (
                    jnp.broadcast_to(v[j], (F_PAD,)))

        pltpu.sync_copy(
            bc,
            out.at[pl.ds((c * NP + s * ROWS_PER_TILE + chunk * DEG_CHUNK)
                         * F_PAD, DEG_CHUNK * F_PAD)])


def _make_agg_kernel(G):
    @functools.partial(
        pl.kernel,
        out_type=jax.ShapeDtypeStruct((2 * G * NP, F_PAD), jnp.float32),
        mesh=_sc_mesh(),
        scratch_types=[
            pltpu.VMEM((2, SB), jnp.int32),
            pltpu.VMEM((2, SB), jnp.int32),
            pltpu.VMEM((2, SB, F_PAD), jnp.float32),
            pltpu.SemaphoreType.DMA,
            pltpu.SemaphoreType.DMA,
            pltpu.SemaphoreType.DMA,
            pltpu.SemaphoreType.DMA,
            pltpu.VMEM_SHARED((NP, F_PAD), jnp.float32),
        ],
        compiler_params=_SC_PARAMS,
    )
    def _agg(src1d, dst1d, zeros2, *rest):
        srcs = rest[:G]
        out = rest[G]
        srcbuf, dstbuf, rows, sg0, sg1, ss0, ss1, acc = rest[G + 1:]
        sem_g = (sg0, sg1)
        sem_s = (ss0, ss1)
        c = lax.axis_index("c")
        s = lax.axis_index("s")
        w = c * NS + s

        def start_scatters(g, o):
            pltpu.async_copy(rows.at[o], acc.at[dstbuf.at[o]], sem_s[o],
                             add=True)

        def drain_gather(g, o):
            pltpu.make_async_copy(srcs[g].at[pl.ds(0, SB)], rows.at[o],
                                  sem_g[o]).wait()

        def drain_scatter(g, k):
            pltpu.make_async_copy(srcs[g].at[pl.ds(0, SB)], rows.at[k],
                                  sem_s[k]).wait()

        for g in range(G):
            pltpu.sync_copy(
                zeros2.at[pl.ds(s * ROWS_PER_TILE, ROWS_PER_TILE)],
                acc.at[pl.ds(s * ROWS_PER_TILE, ROWS_PER_TILE)])
            plsc.subcore_barrier()

            @pl.loop(0, SBPW // 2)
            def _pair(t):
                for k in range(2):
                    sb = t * 2 + k
                    o = 1 - k

                    @pl.when(t >= 1)
                    def _drain_s():
                        drain_scatter(g, k)

                    e0 = w * EPW + sb * SB
                    pltpu.sync_copy(src1d.at[pl.ds(e0, SB)], srcbuf.at[k])
                    pltpu.sync_copy(dst1d.at[pl.ds(e0, SB)], dstbuf.at[k])
                    pltpu.async_copy(srcs[g].at[srcbuf.at[k]], rows.at[k],
                                     sem_g[k])

                    if k == 0:
                        @pl.when(t >= 1)
                        def _pipe():
                            drain_gather(g, o)
                            start_scatters(g, o)
                    else:
                        drain_gather(g, o)
                        start_scatters(g, o)

            drain_gather(g, 1)
            start_scatters(g, 1)
            drain_scatter(g, 0)
            drain_scatter(g, 1)

            plsc.subcore_barrier()
            pltpu.sync_copy(
                acc.at[pl.ds(s * ROWS_PER_TILE, ROWS_PER_TILE)],
                out.at[pl.ds((c * G + g) * NP + s * ROWS_PER_TILE,
                             ROWS_PER_TILE)])
            plsc.subcore_barrier()

    return _agg


_agg1_kernel = _make_agg_kernel(1)
_agg4_kernel = _make_agg_kernel(4)


R = NP // 8
BB = 1600
GRID = R // BB


def _stage1_body(deg_ref, xpad_ref, xs_ref, dinv_ref):
    d = deg_ref[0] + deg_ref[1] + 1.0
    dv = lax.rsqrt(d)
    dinv_ref[...] = dv
    xs_ref[...] = xpad_ref[...] * dv


def _stage1(deg16p, xpadp):
    return pl.pallas_call(
        _stage1_body,
        grid=(GRID,),
        in_specs=[
            pl.BlockSpec((2, BB, 128), lambda i: (0, i, 0)),
            pl.BlockSpec((BB, 128), lambda i: (i, 0)),
        ],
        out_specs=[
            pl.BlockSpec((BB, 128), lambda i: (i, 0)),
            pl.BlockSpec((BB, 128), lambda i: (i, 0)),
        ],
        out_shape=[
            jax.ShapeDtypeStruct((R, 128), jnp.float32),
            jax.ShapeDtypeStruct((R, 128), jnp.float32),
        ],
    )(deg16p, xpadp)


def _stage2_body(a1_ref, xs_ref, dinv_ref, w1s_ref, b1c_ref, *out_refs):
    t = (a1_ref[0] + a1_ref[1] + xs_ref[...]) * dinv_ref[...]
    h = jnp.dot(t, w1s_ref[...], preferred_element_type=jnp.float32)
    h = h + b1c_ref[...]
    r = jnp.maximum(h, 0.0)
    dv = dinv_ref[...]
    for g in range(4):
        out_refs[g][...] = r[:, g * 128:(g + 1) * 128] * dv


def _stage2(a1p, xsp, dinvp, w1s, b1c):
    return pl.pallas_call(
        _stage2_body,
        grid=(GRID,),
        in_specs=[
            pl.BlockSpec((2, BB, 128), lambda i: (0, i, 0)),
            pl.BlockSpec((BB, 128), lambda i: (i, 0)),
            pl.BlockSpec((BB, 128), lambda i: (i, 0)),
            pl.BlockSpec((128, 512), lambda i: (0, 0)),
            pl.BlockSpec((1, 512), lambda i: (0, 0)),
        ],
        out_specs=[pl.BlockSpec((BB, 128), lambda i: (i, 0))] * 4,
        out_shape=[jax.ShapeDtypeStruct((R, 128), jnp.float32)] * 4,
    )(a1p, xsp, dinvp, w1s, b1c)


def _stage3_body(a2_ref, h0_ref, h1_ref, h2_ref, h3_ref, dinv_ref,
                 w2s_ref, b2c_ref, wos_ref, boc_ref, sumg_ref, out_ref):
    hs = (h0_ref[...], h1_ref[...], h2_ref[...], h3_ref[...])
    dv = dinv_ref[...]
    t2 = jnp.concatenate(
        [(a2_ref[g] + a2_ref[4 + g] + hs[g]) * dv for g in range(4)], axis=1)
    h2 = jnp.dot(t2, w2s_ref[...], preferred_element_type=jnp.float32)
    h2 = h2 + b2c_ref[...]
    r2 = jnp.maximum(h2, 0.0)
    lg = jnp.dot(r2, wos_ref[...], preferred_element_type=jnp.float32)
    lg = lg + boc_ref[...]
    m = jnp.max(lg, axis=1, keepdims=True)
    e = jnp.exp(lg - m)
    ssum = jnp.dot(e, sumg_ref[...], preferred_element_type=jnp.float32)
    out_ref[...] = e / ssum


def _stage3(a2p, hs4, dinvp, w2s, b2c, wos, boc, sumg):
    return pl.pallas_call(
        _stage3_body,
        grid=(GRID,),
        in_specs=[
            pl.BlockSpec((8, BB, 128), lambda i: (0, i, 0)),
            pl.BlockSpec((BB, 128), lambda i: (i, 0)),
            pl.BlockSpec((BB, 128), lambda i: (i, 0)),
            pl.BlockSpec((BB, 128), lambda i: (i, 0)),
            pl.BlockSpec((BB, 128), lambda i: (i, 0)),
            pl.BlockSpec((BB, 128), lambda i: (i, 0)),
            pl.BlockSpec((512, 512), lambda i: (0, 0)),
            pl.BlockSpec((1, 512), lambda i: (0, 0)),
            pl.BlockSpec((512, 64), lambda i: (0, 0)),
            pl.BlockSpec((1, 64), lambda i: (0, 0)),
            pl.BlockSpec((64, 64), lambda i: (0, 0)),
        ],
        out_specs=pl.BlockSpec((BB, 64), lambda i: (i, 0)),
        out_shape=jax.ShapeDtypeStruct((R, 64), jnp.float32),
    )(a2p, *hs4, dinvp, w2s, b2c, wos, boc, sumg)


def kernel(x, edge_index, batch, W1, b1, W2, b2, Wo, bo):
    del batch
    f_in = x.shape[2]
    eye8 = jnp.eye(8, dtype=jnp.float32)

    x_last = x[:, -1, :]
    xpad = jnp.zeros((NP, F_PAD), jnp.float32).at[:N, :f_in].set(x_last)
    xpadp = xpad.reshape(R, 128)

    pad_idx = (N + (jnp.arange(EP - E, dtype=jnp.int32) % (NP - N)))
    src = jnp.concatenate([edge_index[0], pad_idx])
    dst = jnp.concatenate([edge_index[1], pad_idx])

    zeros1 = jnp.zeros((NP,), jnp.float32)
    zeros2 = jnp.zeros((NP, F_PAD), jnp.float32)
    ones1 = jnp.ones((SB,), jnp.float32)

    w1p = jnp.zeros((F_PAD, H), jnp.float32).at[:f_in, :].set(W1)
    w1s = jnp.concatenate(
        [jnp.kron(eye8, w1p[:, g * 16:(g + 1) * 16]) for g in range(4)],
        axis=1)
    b1c = jnp.tile(b1.reshape(4, 16), (1, 8)).reshape(1, 512)
    w2r = W2.reshape(4, 16, 4, 16)
    w2s = jnp.concatenate(
        [jnp.concatenate([jnp.kron(eye8, w2r[gi, :, go, :])
                          for gi in range(4)], axis=0)
         for go in range(4)], axis=1)
    b2c = jnp.tile(b2.reshape(4, 16), (1, 8)).reshape(1, 512)
    wop = jnp.zeros((H, 8), jnp.float32).at[:, :OUT].set(Wo)
    wos = jnp.concatenate(
        [jnp.kron(eye8, wop[g * 16:(g + 1) * 16, :]) for g in range(4)],
        axis=0)
    bo8 = jnp.full((8,), -1e30, jnp.float32).at[:OUT].set(bo)
    boc = jnp.tile(bo8, 8).reshape(1, 64)
    sumg = jnp.kron(eye8, jnp.ones((8, 8), jnp.float32))

    deg16p = _deg_kernel(dst, zeros1, ones1).reshape(2, R, 128)

    xsp, dinvp = _stage1(deg16p, xpadp)

    a1 = _agg1_kernel(src, dst, zeros2, xsp.reshape(NP, F_PAD))
    a1p = a1.reshape(2, R, 128)

    hs4 = _stage2(a1p, xsp, dinvp, w1s, b1c)

    a2 = _agg4_kernel(src, dst, zeros2,
                      *[h.reshape(NP, F_PAD) for h in hs4])
    a2p = a2.reshape(8, R, 128)

    probs = _stage3(a2p, hs4, dinvp, w2s, b2c, wos, boc, sumg)

    return probs.reshape(NP, 8)[:N, :OUT]


# speedup vs baseline: 49.5639x; 1.3222x over previous
import functools

import jax
import jax.numpy as jnp
from jax import lax
from jax.experimental import pallas as pl
from jax.experimental.pallas import tpu as pltpu
from jax.experimental.pallas import tpu_sc as plsc

N = 100000
E = 1600000
F_PAD = 16
H = 64
OUT = 5

NP = 102400
NC = 2
NS = 16
NW = NC * NS
SB = 512
SBB = SB // 128
SBPW = 100
EPW = SBPW * SB
EP = NW * EPW
ROWS_PER_TILE = NP // NS


def _sc_mesh():
    return plsc.VectorSubcoreMesh(core_axis_name="c", subcore_axis_name="s")


_SC_PARAMS = pltpu.CompilerParams(use_tc_tiling_on_sc=False)


DEG_CHUNK = 800


@functools.partial(
    pl.kernel,
    out_type=jax.ShapeDtypeStruct((2 * NP * F_PAD,), jnp.float32),
    mesh=_sc_mesh(),
    scratch_types=[
        pltpu.VMEM((2, SB), jnp.int32),
        pltpu.VMEM((SB,), jnp.float32),
        pltpu.VMEM((ROWS_PER_TILE,), jnp.float32),
        pltpu.VMEM((DEG_CHUNK * F_PAD,), jnp.float32),
        pltpu.SemaphoreType.DMA,
        pltpu.SemaphoreType.DMA,
        pltpu.VMEM_SHARED((NP,), jnp.float32),
    ],
    compiler_params=_SC_PARAMS,
)
def _deg_kernel(dst1d, zeros1, ones1, out, dstbuf, onesbuf, degv, bc,
                sem0, sem1, acc):
    c = lax.axis_index("c")
    s = lax.axis_index("s")
    w = c * NS + s
    sems = (sem0, sem1)

    pltpu.sync_copy(zeros1.at[pl.ds(s * ROWS_PER_TILE, ROWS_PER_TILE)],
                    acc.at[pl.ds(s * ROWS_PER_TILE, ROWS_PER_TILE)])
    pltpu.sync_copy(ones1, onesbuf)
    plsc.subcore_barrier()

    @pl.loop(0, SBPW // 2)
    def _pair(t):
        for k in range(2):
            sb = t * 2 + k

            @pl.when(t >= 1)
            def _drain():
                pltpu.make_async_copy(dst1d.at[pl.ds(0, SB)],
                                      dstbuf.at[k], sems[k]).wait()

            e0 = w * EPW + sb * SB
            pltpu.sync_copy(dst1d.at[pl.ds(e0, SB)], dstbuf.at[k])
            pltpu.async_copy(onesbuf, acc.at[dstbuf.at[k]], sems[k],
                             add=True)

    for k in range(2):
        pltpu.make_async_copy(dst1d.at[pl.ds(0, SB)], dstbuf.at[k],
                              sems[k]).wait()

    plsc.subcore_barrier()
    pltpu.sync_copy(acc.at[pl.ds(s * ROWS_PER_TILE, ROWS_PER_TILE)], degv)
    for chunk in range(ROWS_PER_TILE // DEG_CHUNK):
        @pl.loop(0, DEG_CHUNK // 16)
        def _bcast(i):
            v = degv[pl.ds(chunk * DEG_CHUNK + i * 16, 16)]
            for j in range(16):
                bc[pl.ds((i * 16 + j) * F_PAD, F_PAD)] = (
                    jnp.broadcast_to(v[j], (F_PAD,)))

        pltpu.sync_copy(
            bc,
            out.at[pl.ds((c * NP + s * ROWS_PER_TILE + chunk * DEG_CHUNK)
                         * F_PAD, DEG_CHUNK * F_PAD)])


def _make_agg_kernel(G):
    @functools.partial(
        pl.kernel,
        out_type=jax.ShapeDtypeStruct((2 * G * NP, F_PAD), jnp.float32),
        mesh=_sc_mesh(),
        scratch_types=[
            pltpu.VMEM((4, SB), jnp.int32),
            pltpu.VMEM((4, SB), jnp.int32),
            pltpu.VMEM((2, SB, F_PAD), jnp.float32),
            [pltpu.SemaphoreType.DMA] * 2,
            [pltpu.SemaphoreType.DMA] * 2,
            [pltpu.SemaphoreType.DMA] * 4,
            pltpu.VMEM_SHARED((NP, F_PAD), jnp.float32),
        ],
        compiler_params=_SC_PARAMS,
    )
    def _agg(src1d, dst1d, zeros2, *rest):
        srcs = rest[:G]
        out = rest[G]
        srcbuf, dstbuf, rows, sem_g, sem_s, sem_i, acc = rest[G + 1:]
        c = lax.axis_index("c")
        s = lax.axis_index("s")
        w = c * NS + s

        def load_idx(e0, q, sem=None):
            if sem is None:
                pltpu.sync_copy(src1d.at[pl.ds(e0, SB)], srcbuf.at[q])
                pltpu.sync_copy(dst1d.at[pl.ds(e0, SB)], dstbuf.at[q])
            else:
                pltpu.async_copy(src1d.at[pl.ds(e0, SB)], srcbuf.at[q], sem)
                pltpu.async_copy(dst1d.at[pl.ds(e0, SB)], dstbuf.at[q], sem)

        def drain_idx(q):
            pltpu.make_async_copy(src1d.at[pl.ds(0, SB)], srcbuf.at[q],
                                  sem_i[q]).wait()
            pltpu.make_async_copy(dst1d.at[pl.ds(0, SB)], dstbuf.at[q],
                                  sem_i[q]).wait()

        def start_scatter(o, q):
            pltpu.async_copy(rows.at[o], acc.at[dstbuf.at[q]], sem_s[o],
                             add=True)

        def drain_rows(g, sem):
            pltpu.make_async_copy(srcs[g].at[pl.ds(0, SB)], rows.at[0],
                                  sem).wait()

        for g in range(G):
            pltpu.sync_copy(
                zeros2.at[pl.ds(s * ROWS_PER_TILE, ROWS_PER_TILE)],
                acc.at[pl.ds(s * ROWS_PER_TILE, ROWS_PER_TILE)])
            plsc.subcore_barrier()

            base = w * EPW
            load_idx(base, 0)
            load_idx(base + SB, 1)

            @pl.loop(0, SBPW // 4)
            def _quad(t):
                for j in range(4):
                    k = j % 2
                    o = 1 - k
                    q = j

                    def _step(t=t, j=j, k=k, o=o, q=q):
                        sb = t * 4 + j
                        drain_rows(g, sem_s[k])
                        if j < 2:
                            @pl.when(t >= 1)
                            def _di():
                                drain_idx(q)
                        else:
                            drain_idx(q)
                        pltpu.async_copy(srcs[g].at[srcbuf.at[q]],
                                         rows.at[k], sem_g[k])
                        if j < 2:
                            load_idx(w * EPW + (sb + 2) * SB, (q + 2) % 4,
                                     sem_i[(q + 2) % 4])
                        else:
                            @pl.when(t < SBPW // 4 - 1)
                            def _pf():
                                load_idx(w * EPW + (sb + 2) * SB,
                                         (q + 2) % 4, sem_i[(q + 2) % 4])
                        drain_rows(g, sem_g[o])
                        start_scatter(o, (q + 3) % 4)

                    if j == 0:
                        @pl.when(t >= 1)
                        def _s0():
                            _step()

                        @pl.when(t < 1)
                        def _s0p():
                            pltpu.async_copy(srcs[g].at[srcbuf.at[0]],
                                             rows.at[0], sem_g[0])
                            load_idx(w * EPW + 2 * SB, 2, sem_i[2])
                    elif j == 1:
                        @pl.when(t >= 1)
                        def _s1():
                            _step()

                        @pl.when(t < 1)
                        def _s1p():
                            pltpu.async_copy(srcs[g].at[srcbuf.at[1]],
                                             rows.at[1], sem_g[1])
                            load_idx(w * EPW + 3 * SB, 3, sem_i[3])
                            drain_rows(g, sem_g[0])
                            start_scatter(0, 0)
                    else:
                        _step()

            drain_rows(g, sem_g[1])
            start_scatter(1, 3)
            drain_rows(g, sem_s[0])
            drain_rows(g, sem_s[1])

            plsc.subcore_barrier()
            pltpu.sync_copy(
                acc.at[pl.ds(s * ROWS_PER_TILE, ROWS_PER_TILE)],
                out.at[pl.ds((c * G + g) * NP + s * ROWS_PER_TILE,
                             ROWS_PER_TILE)])
            plsc.subcore_barrier()

    return _agg


_agg1_kernel = _make_agg_kernel(1)
_agg4_kernel = _make_agg_kernel(4)


R = NP // 8
BB = 1600
GRID = R // BB


def _stage1_body(deg_ref, xpad_ref, xs_ref, dinv_ref):
    d = deg_ref[0] + deg_ref[1] + 1.0
    dv = lax.rsqrt(d)
    dinv_ref[...] = dv
    xs_ref[...] = xpad_ref[...] * dv


def _stage1(deg16p, xpadp):
    return pl.pallas_call(
        _stage1_body,
        grid=(GRID,),
        in_specs=[
            pl.BlockSpec((2, BB, 128), lambda i: (0, i, 0)),
            pl.BlockSpec((BB, 128), lambda i: (i, 0)),
        ],
        out_specs=[
            pl.BlockSpec((BB, 128), lambda i: (i, 0)),
            pl.BlockSpec((BB, 128), lambda i: (i, 0)),
        ],
        out_shape=[
            jax.ShapeDtypeStruct((R, 128), jnp.float32),
            jax.ShapeDtypeStruct((R, 128), jnp.float32),
        ],
    )(deg16p, xpadp)


def _stage2_body(a1_ref, xs_ref, dinv_ref, w1s_ref, b1c_ref, *out_refs):
    t = (a1_ref[0] + a1_ref[1] + xs_ref[...]) * dinv_ref[...]
    h = jnp.dot(t, w1s_ref[...], preferred_element_type=jnp.float32)
    h = h + b1c_ref[...]
    r = jnp.maximum(h, 0.0)
    dv = dinv_ref[...]
    for g in range(4):
        out_refs[g][...] = r[:, g * 128:(g + 1) * 128] * dv


def _stage2(a1p, xsp, dinvp, w1s, b1c):
    return pl.pallas_call(
        _stage2_body,
        grid=(GRID,),
        in_specs=[
            pl.BlockSpec((2, BB, 128), lambda i: (0, i, 0)),
            pl.BlockSpec((BB, 128), lambda i: (i, 0)),
            pl.BlockSpec((BB, 128), lambda i: (i, 0)),
            pl.BlockSpec((128, 512), lambda i: (0, 0)),
            pl.BlockSpec((1, 512), lambda i: (0, 0)),
        ],
        out_specs=[pl.BlockSpec((BB, 128), lambda i: (i, 0))] * 4,
        out_shape=[jax.ShapeDtypeStruct((R, 128), jnp.float32)] * 4,
    )(a1p, xsp, dinvp, w1s, b1c)


def _stage3_body(a2_ref, h0_ref, h1_ref, h2_ref, h3_ref, dinv_ref,
                 w2s_ref, b2c_ref, wos_ref, boc_ref, sumg_ref, out_ref):
    hs = (h0_ref[...], h1_ref[...], h2_ref[...], h3_ref[...])
    dv = dinv_ref[...]
    t2 = jnp.concatenate(
        [(a2_ref[g] + a2_ref[4 + g] + hs[g]) * dv for g in range(4)], axis=1)
    h2 = jnp.dot(t2, w2s_ref[...], preferred_element_type=jnp.float32)
    h2 = h2 + b2c_ref[...]
    r2 = jnp.maximum(h2, 0.0)
    lg = jnp.dot(r2, wos_ref[...], preferred_element_type=jnp.float32)
    lg = lg + boc_ref[...]
    m = jnp.max(lg, axis=1, keepdims=True)
    e = jnp.exp(lg - m)
    ssum = jnp.dot(e, sumg_ref[...], preferred_element_type=jnp.float32)
    out_ref[...] = e / ssum


def _stage3(a2p, hs4, dinvp, w2s, b2c, wos, boc, sumg):
    return pl.pallas_call(
        _stage3_body,
        grid=(GRID,),
        in_specs=[
            pl.BlockSpec((8, BB, 128), lambda i: (0, i, 0)),
            pl.BlockSpec((BB, 128), lambda i: (i, 0)),
            pl.BlockSpec((BB, 128), lambda i: (i, 0)),
            pl.BlockSpec((BB, 128), lambda i: (i, 0)),
            pl.BlockSpec((BB, 128), lambda i: (i, 0)),
            pl.BlockSpec((BB, 128), lambda i: (i, 0)),
            pl.BlockSpec((512, 512), lambda i: (0, 0)),
            pl.BlockSpec((1, 512), lambda i: (0, 0)),
            pl.BlockSpec((512, 64), lambda i: (0, 0)),
            pl.BlockSpec((1, 64), lambda i: (0, 0)),
            pl.BlockSpec((64, 64), lambda i: (0, 0)),
        ],
        out_specs=pl.BlockSpec((BB, 64), lambda i: (i, 0)),
        out_shape=jax.ShapeDtypeStruct((R, 64), jnp.float32),
    )(a2p, *hs4, dinvp, w2s, b2c, wos, boc, sumg)


def kernel(x, edge_index, batch, W1, b1, W2, b2, Wo, bo):
    del batch
    f_in = x.shape[2]
    eye8 = jnp.eye(8, dtype=jnp.float32)

    x_last = x[:, -1, :]
    xpad = jnp.zeros((NP, F_PAD), jnp.float32).at[:N, :f_in].set(x_last)
    xpadp = xpad.reshape(R, 128)

    pad_idx = (N + (jnp.arange(EP - E, dtype=jnp.int32) % (NP - N)))
    src = jnp.concatenate([edge_index[0], pad_idx])
    dst = jnp.concatenate([edge_index[1], pad_idx])

    zeros1 = jnp.zeros((NP,), jnp.float32)
    zeros2 = jnp.zeros((NP, F_PAD), jnp.float32)
    ones1 = jnp.ones((SB,), jnp.float32)

    w1p = jnp.zeros((F_PAD, H), jnp.float32).at[:f_in, :].set(W1)
    w1s = jnp.concatenate(
        [jnp.kron(eye8, w1p[:, g * 16:(g + 1) * 16]) for g in range(4)],
        axis=1)
    b1c = jnp.tile(b1.reshape(4, 16), (1, 8)).reshape(1, 512)
    w2r = W2.reshape(4, 16, 4, 16)
    w2s = jnp.concatenate(
        [jnp.concatenate([jnp.kron(eye8, w2r[gi, :, go, :])
                          for gi in range(4)], axis=0)
         for go in range(4)], axis=1)
    b2c = jnp.tile(b2.reshape(4, 16), (1, 8)).reshape(1, 512)
    wop = jnp.zeros((H, 8), jnp.float32).at[:, :OUT].set(Wo)
    wos = jnp.concatenate(
        [jnp.kron(eye8, wop[g * 16:(g + 1) * 16, :]) for g in range(4)],
        axis=0)
    bo8 = jnp.full((8,), -1e30, jnp.float32).at[:OUT].set(bo)
    boc = jnp.tile(bo8, 8).reshape(1, 64)
    sumg = jnp.kron(eye8, jnp.ones((8, 8), jnp.float32))

    deg16p = _deg_kernel(dst, zeros1, ones1).reshape(2, R, 128)

    xsp, dinvp = _stage1(deg16p, xpadp)

    a1 = _agg1_kernel(src, dst, zeros2, xsp.reshape(NP, F_PAD))
    a1p = a1.reshape(2, R, 128)

    hs4 = _stage2(a1p, xsp, dinvp, w1s, b1c)

    a2 = _agg4_kernel(src, dst, zeros2,
                      *[h.reshape(NP, F_PAD) for h in hs4])
    a2p = a2.reshape(8, R, 128)

    probs = _stage3(a2p, hs4, dinvp, w2s, b2c, wos, boc, sumg)

    return probs.reshape(NP, 8)[:N, :OUT]


# speedup vs baseline: 50.1025x; 1.0109x over previous
import functools

import jax
import jax.numpy as jnp
from jax import lax
from jax.experimental import pallas as pl
from jax.experimental.pallas import tpu as pltpu
from jax.experimental.pallas import tpu_sc as plsc

N = 100000
E = 1600000
F_PAD = 16
H = 64
OUT = 5

NP = 102400
NC = 2
NS = 16
NW = NC * NS
SB = 512
SBB = SB // 128
SBPW = 100
EPW = SBPW * SB
EP = NW * EPW
ROWS_PER_TILE = NP // NS


def _sc_mesh():
    return plsc.VectorSubcoreMesh(core_axis_name="c", subcore_axis_name="s")


_SC_PARAMS = pltpu.CompilerParams(use_tc_tiling_on_sc=False)


DEG_CHUNK = 800


@functools.partial(
    pl.kernel,
    out_type=jax.ShapeDtypeStruct((2 * NP * F_PAD,), jnp.float32),
    mesh=_sc_mesh(),
    scratch_types=[
        pltpu.VMEM((4, SB), jnp.int32),
        pltpu.VMEM((SB,), jnp.float32),
        pltpu.VMEM((ROWS_PER_TILE,), jnp.float32),
        pltpu.VMEM((DEG_CHUNK * F_PAD,), jnp.float32),
        [pltpu.SemaphoreType.DMA] * 2,
        [pltpu.SemaphoreType.DMA] * 4,
        pltpu.VMEM_SHARED((NP,), jnp.float32),
    ],
    compiler_params=_SC_PARAMS,
)
def _deg_kernel(dst1d, zeros1, ones1, out, dstbuf, onesbuf, degv, bc,
                sem_s, sem_i, acc):
    c = lax.axis_index("c")
    s = lax.axis_index("s")
    w = c * NS + s

    def drain_2kb(sem):
        pltpu.make_async_copy(dst1d.at[pl.ds(0, SB)], dstbuf.at[0],
                              sem).wait()

    pltpu.sync_copy(zeros1.at[pl.ds(s * ROWS_PER_TILE, ROWS_PER_TILE)],
                    acc.at[pl.ds(s * ROWS_PER_TILE, ROWS_PER_TILE)])
    pltpu.sync_copy(ones1, onesbuf)
    plsc.subcore_barrier()

    base = w * EPW
    pltpu.sync_copy(dst1d.at[pl.ds(base, SB)], dstbuf.at[0])
    pltpu.sync_copy(dst1d.at[pl.ds(base + SB, SB)], dstbuf.at[1])

    @pl.loop(0, SBPW // 4)
    def _quad(t):
        for j in range(4):
            k = j % 2
            q = j

            def _step(t=t, j=j, k=k, q=q):
                sb = t * 4 + j
                drain_2kb(sem_s[k])
                if j < 2:
                    @pl.when(t >= 1)
                    def _di():
                        drain_2kb(sem_i[q])
                else:
                    drain_2kb(sem_i[q])
                pltpu.async_copy(onesbuf, acc.at[dstbuf.at[q]], sem_s[k],
                                 add=True)
                if j < 2:
                    pltpu.async_copy(dst1d.at[pl.ds(base + (sb + 2) * SB,
                                                    SB)],
                                     dstbuf.at[(q + 2) % 4],
                                     sem_i[(q + 2) % 4])
                else:
                    @pl.when(t < SBPW // 4 - 1)
                    def _pf():
                        pltpu.async_copy(dst1d.at[pl.ds(base + (sb + 2) * SB,
                                                        SB)],
                                         dstbuf.at[(q + 2) % 4],
                                         sem_i[(q + 2) % 4])

            if j < 2:
                @pl.when(t >= 1)
                def _ss():
                    _step()

                @pl.when(t < 1)
                def _sp(j=j, k=k, q=q):
                    pltpu.async_copy(onesbuf, acc.at[dstbuf.at[q]],
                                     sem_s[k], add=True)
                    pltpu.async_copy(dst1d.at[pl.ds(base + (j + 2) * SB,
                                                    SB)],
                                     dstbuf.at[q + 2], sem_i[q + 2])
            else:
                _step()

    for k in range(2):
        drain_2kb(sem_s[k])

    plsc.subcore_barrier()
    pltpu.sync_copy(acc.at[pl.ds(s * ROWS_PER_TILE, ROWS_PER_TILE)], degv)
    for chunk in range(ROWS_PER_TILE // DEG_CHUNK):
        @pl.loop(0, DEG_CHUNK // 16)
        def _bcast(i):
            v = degv[pl.ds(chunk * DEG_CHUNK + i * 16, 16)]
            for j in range(16):
                bc[pl.ds((i * 16 + j) * F_PAD, F_PAD)] = (
                    jnp.broadcast_to(v[j], (F_PAD,)))

        pltpu.sync_copy(
            bc,
            out.at[pl.ds((c * NP + s * ROWS_PER_TILE + chunk * DEG_CHUNK)
                         * F_PAD, DEG_CHUNK * F_PAD)])


def _make_agg_kernel(G):
    @functools.partial(
        pl.kernel,
        out_type=jax.ShapeDtypeStruct((2 * G * NP, F_PAD), jnp.float32),
        mesh=_sc_mesh(),
        scratch_types=[
            pltpu.VMEM((4, SB), jnp.int32),
            pltpu.VMEM((4, SB), jnp.int32),
            pltpu.VMEM((2, SB, F_PAD), jnp.float32),
            [pltpu.SemaphoreType.DMA] * 2,
            [pltpu.SemaphoreType.DMA] * 2,
            [pltpu.SemaphoreType.DMA] * 4,
            pltpu.VMEM_SHARED((NP, F_PAD), jnp.float32),
        ],
        compiler_params=_SC_PARAMS,
    )
    def _agg(src1d, dst1d, zeros2, *rest):
        srcs = rest[:G]
        out = rest[G]
        srcbuf, dstbuf, rows, sem_g, sem_s, sem_i, acc = rest[G + 1:]
        c = lax.axis_index("c")
        s = lax.axis_index("s")
        w = c * NS + s

        def load_idx(e0, q, sem=None):
            if sem is None:
                pltpu.sync_copy(src1d.at[pl.ds(e0, SB)], srcbuf.at[q])
                pltpu.sync_copy(dst1d.at[pl.ds(e0, SB)], dstbuf.at[q])
            else:
                pltpu.async_copy(src1d.at[pl.ds(e0, SB)], srcbuf.at[q], sem)
                pltpu.async_copy(dst1d.at[pl.ds(e0, SB)], dstbuf.at[q], sem)

        def drain_idx(q):
            pltpu.make_async_copy(src1d.at[pl.ds(0, SB)], srcbuf.at[q],
                                  sem_i[q]).wait()
            pltpu.make_async_copy(dst1d.at[pl.ds(0, SB)], dstbuf.at[q],
                                  sem_i[q]).wait()

        def start_scatter(o, q):
            pltpu.async_copy(rows.at[o], acc.at[dstbuf.at[q]], sem_s[o],
                             add=True)

        def drain_rows(g, sem):
            pltpu.make_async_copy(srcs[g].at[pl.ds(0, SB)], rows.at[0],
                                  sem).wait()

        for g in range(G):
            pltpu.sync_copy(
                zeros2.at[pl.ds(s * ROWS_PER_TILE, ROWS_PER_TILE)],
                acc.at[pl.ds(s * ROWS_PER_TILE, ROWS_PER_TILE)])
            plsc.subcore_barrier()

            base = w * EPW
            load_idx(base, 0)
            load_idx(base + SB, 1)

            @pl.loop(0, SBPW // 4)
            def _quad(t):
                for j in range(4):
                    k = j % 2
                    o = 1 - k
                    q = j

                    def _step(t=t, j=j, k=k, o=o, q=q):
                        sb = t * 4 + j
                        drain_rows(g, sem_s[k])
                        if j < 2:
                            @pl.when(t >= 1)
                            def _di():
                                drain_idx(q)
                        else:
                            drain_idx(q)
                        pltpu.async_copy(srcs[g].at[srcbuf.at[q]],
                                         rows.at[k], sem_g[k])
                        if j < 2:
                            load_idx(w * EPW + (sb + 2) * SB, (q + 2) % 4,
                                     sem_i[(q + 2) % 4])
                        else:
                            @pl.when(t < SBPW // 4 - 1)
                            def _pf():
                                load_idx(w * EPW + (sb + 2) * SB,
                                         (q + 2) % 4, sem_i[(q + 2) % 4])
                        drain_rows(g, sem_g[o])
                        start_scatter(o, (q + 3) % 4)

                    if j == 0:
                        @pl.when(t >= 1)
                        def _s0():
                            _step()

                        @pl.when(t < 1)
                        def _s0p():
                            pltpu.async_copy(srcs[g].at[srcbuf.at[0]],
                                             rows.at[0], sem_g[0])
                            load_idx(w * EPW + 2 * SB, 2, sem_i[2])
                    elif j == 1:
                        @pl.when(t >= 1)
                        def _s1():
                            _step()

                        @pl.when(t < 1)
                        def _s1p():
                            pltpu.async_copy(srcs[g].at[srcbuf.at[1]],
                                             rows.at[1], sem_g[1])
                            load_idx(w * EPW + 3 * SB, 3, sem_i[3])
                            drain_rows(g, sem_g[0])
                            start_scatter(0, 0)
                    else:
                        _step()

            drain_rows(g, sem_g[1])
            start_scatter(1, 3)
            drain_rows(g, sem_s[0])
            drain_rows(g, sem_s[1])

            plsc.subcore_barrier()
            pltpu.sync_copy(
                acc.at[pl.ds(s * ROWS_PER_TILE, ROWS_PER_TILE)],
                out.at[pl.ds((c * G + g) * NP + s * ROWS_PER_TILE,
                             ROWS_PER_TILE)])
            plsc.subcore_barrier()

    return _agg


_agg1_kernel = _make_agg_kernel(1)
_agg4_kernel = _make_agg_kernel(4)


R = NP // 8
BB = 1600
GRID = R // BB


def _stage1_body(deg_ref, xpad_ref, xs_ref, dinv_ref):
    d = deg_ref[0] + deg_ref[1] + 1.0
    dv = lax.rsqrt(d)
    dinv_ref[...] = dv
    xs_ref[...] = xpad_ref[...] * dv


def _stage1(deg16p, xpadp):
    return pl.pallas_call(
        _stage1_body,
        grid=(GRID,),
        in_specs=[
            pl.BlockSpec((2, BB, 128), lambda i: (0, i, 0)),
            pl.BlockSpec((BB, 128), lambda i: (i, 0)),
        ],
        out_specs=[
            pl.BlockSpec((BB, 128), lambda i: (i, 0)),
            pl.BlockSpec((BB, 128), lambda i: (i, 0)),
        ],
        out_shape=[
            jax.ShapeDtypeStruct((R, 128), jnp.float32),
            jax.ShapeDtypeStruct((R, 128), jnp.float32),
        ],
    )(deg16p, xpadp)


def _stage2_body(a1_ref, xs_ref, dinv_ref, w1s_ref, b1c_ref, *out_refs):
    t = (a1_ref[0] + a1_ref[1] + xs_ref[...]) * dinv_ref[...]
    h = jnp.dot(t, w1s_ref[...], preferred_element_type=jnp.float32)
    h = h + b1c_ref[...]
    r = jnp.maximum(h, 0.0)
    dv = dinv_ref[...]
    for g in range(4):
        out_refs[g][...] = r[:, g * 128:(g + 1) * 128] * dv


def _stage2(a1p, xsp, dinvp, w1s, b1c):
    return pl.pallas_call(
        _stage2_body,
        grid=(GRID,),
        in_specs=[
            pl.BlockSpec((2, BB, 128), lambda i: (0, i, 0)),
            pl.BlockSpec((BB, 128), lambda i: (i, 0)),
            pl.BlockSpec((BB, 128), lambda i: (i, 0)),
            pl.BlockSpec((128, 512), lambda i: (0, 0)),
            pl.BlockSpec((1, 512), lambda i: (0, 0)),
        ],
        out_specs=[pl.BlockSpec((BB, 128), lambda i: (i, 0))] * 4,
        out_shape=[jax.ShapeDtypeStruct((R, 128), jnp.float32)] * 4,
    )(a1p, xsp, dinvp, w1s, b1c)


def _stage3_body(a2_ref, h0_ref, h1_ref, h2_ref, h3_ref, dinv_ref,
                 w2s_ref, b2c_ref, wos_ref, boc_ref, sumg_ref, out_ref):
    hs = (h0_ref[...], h1_ref[...], h2_ref[...], h3_ref[...])
    dv = dinv_ref[...]
    t2 = jnp.concatenate(
        [(a2_ref[g] + a2_ref[4 + g] + hs[g]) * dv for g in range(4)], axis=1)
    h2 = jnp.dot(t2, w2s_ref[...], preferred_element_type=jnp.float32)
    h2 = h2 + b2c_ref[...]
    r2 = jnp.maximum(h2, 0.0)
    lg = jnp.dot(r2, wos_ref[...], preferred_element_type=jnp.float32)
    lg = lg + boc_ref[...]
    m = jnp.max(lg, axis=1, keepdims=True)
    e = jnp.exp(lg - m)
    ssum = jnp.dot(e, sumg_ref[...], preferred_element_type=jnp.float32)
    out_ref[...] = e / ssum


def _stage3(a2p, hs4, dinvp, w2s, b2c, wos, boc, sumg):
    return pl.pallas_call(
        _stage3_body,
        grid=(GRID,),
        in_specs=[
            pl.BlockSpec((8, BB, 128), lambda i: (0, i, 0)),
            pl.BlockSpec((BB, 128), lambda i: (i, 0)),
            pl.BlockSpec((BB, 128), lambda i: (i, 0)),
            pl.BlockSpec((BB, 128), lambda i: (i, 0)),
            pl.BlockSpec((BB, 128), lambda i: (i, 0)),
            pl.BlockSpec((BB, 128), lambda i: (i, 0)),
            pl.BlockSpec((512, 512), lambda i: (0, 0)),
            pl.BlockSpec((1, 512), lambda i: (0, 0)),
            pl.BlockSpec((512, 64), lambda i: (0, 0)),
            pl.BlockSpec((1, 64), lambda i: (0, 0)),
            pl.BlockSpec((64, 64), lambda i: (0, 0)),
        ],
        out_specs=pl.BlockSpec((BB, 64), lambda i: (i, 0)),
        out_shape=jax.ShapeDtypeStruct((R, 64), jnp.float32),
    )(a2p, *hs4, dinvp, w2s, b2c, wos, boc, sumg)


def kernel(x, edge_index, batch, W1, b1, W2, b2, Wo, bo):
    del batch
    f_in = x.shape[2]
    eye8 = jnp.eye(8, dtype=jnp.float32)

    x_last = x[:, -1, :]
    xpad = jnp.zeros((NP, F_PAD), jnp.float32).at[:N, :f_in].set(x_last)
    xpadp = xpad.reshape(R, 128)

    pad_idx = (N + (jnp.arange(EP - E, dtype=jnp.int32) % (NP - N)))
    src = jnp.concatenate([edge_index[0], pad_idx])
    dst = jnp.concatenate([edge_index[1], pad_idx])

    zeros1 = jnp.zeros((NP,), jnp.float32)
    zeros2 = jnp.zeros((NP, F_PAD), jnp.float32)
    ones1 = jnp.ones((SB,), jnp.float32)

    w1p = jnp.zeros((F_PAD, H), jnp.float32).at[:f_in, :].set(W1)
    w1s = jnp.concatenate(
        [jnp.kron(eye8, w1p[:, g * 16:(g + 1) * 16]) for g in range(4)],
        axis=1)
    b1c = jnp.tile(b1.reshape(4, 16), (1, 8)).reshape(1, 512)
    w2r = W2.reshape(4, 16, 4, 16)
    w2s = jnp.concatenate(
        [jnp.concatenate([jnp.kron(eye8, w2r[gi, :, go, :])
                          for gi in range(4)], axis=0)
         for go in range(4)], axis=1)
    b2c = jnp.tile(b2.reshape(4, 16), (1, 8)).reshape(1, 512)
    wop = jnp.zeros((H, 8), jnp.float32).at[:, :OUT].set(Wo)
    wos = jnp.concatenate(
        [jnp.kron(eye8, wop[g * 16:(g + 1) * 16, :]) for g in range(4)],
        axis=0)
    bo8 = jnp.full((8,), -1e30, jnp.float32).at[:OUT].set(bo)
    boc = jnp.tile(bo8, 8).reshape(1, 64)
    sumg = jnp.kron(eye8, jnp.ones((8, 8), jnp.float32))

    deg16p = _deg_kernel(dst, zeros1, ones1).reshape(2, R, 128)

    xsp, dinvp = _stage1(deg16p, xpadp)

    a1 = _agg1_kernel(src, dst, zeros2, xsp.reshape(NP, F_PAD))
    a1p = a1.reshape(2, R, 128)

    hs4 = _stage2(a1p, xsp, dinvp, w1s, b1c)

    a2 = _agg4_kernel(src, dst, zeros2,
                      *[h.reshape(NP, F_PAD) for h in hs4])
    a2p = a2.reshape(8, R, 128)

    probs = _stage3(a2p, hs4, dinvp, w2s, b2c, wos, boc, sumg)

    return probs.reshape(NP, 8)[:N, :OUT]


# speedup vs baseline: 50.9978x; 1.0179x over previous
import functools

import jax
import jax.numpy as jnp
from jax import lax
from jax.experimental import pallas as pl
from jax.experimental.pallas import tpu as pltpu
from jax.experimental.pallas import tpu_sc as plsc

N = 100000
E = 1600000
F_PAD = 16
H = 64
OUT = 5

NP = 100096
NC = 2
NS = 16
NW = NC * NS
SB = 640
SBPW = 80
EPW = SBPW * SB
EP = NW * EPW
ROWS_PER_TILE = NP // NS


def _sc_mesh():
    return plsc.VectorSubcoreMesh(core_axis_name="c", subcore_axis_name="s")


_SC_PARAMS = pltpu.CompilerParams(use_tc_tiling_on_sc=False)


DEG_CHUNK = 368


@functools.partial(
    pl.kernel,
    out_type=jax.ShapeDtypeStruct((2 * NP * F_PAD,), jnp.float32),
    mesh=_sc_mesh(),
    scratch_types=[
        pltpu.VMEM((4, SB), jnp.int32),
        pltpu.VMEM((SB,), jnp.float32),
        pltpu.VMEM((ROWS_PER_TILE,), jnp.float32),
        pltpu.VMEM((DEG_CHUNK * F_PAD,), jnp.float32),
        [pltpu.SemaphoreType.DMA] * 2,
        [pltpu.SemaphoreType.DMA] * 4,
        pltpu.VMEM_SHARED((NP,), jnp.float32),
    ],
    compiler_params=_SC_PARAMS,
)
def _deg_kernel(dst1d, zeros1, ones1, out, dstbuf, onesbuf, degv, bc,
                sem_s, sem_i, acc):
    c = lax.axis_index("c")
    s = lax.axis_index("s")
    w = c * NS + s

    def drain_2kb(sem):
        pltpu.make_async_copy(dst1d.at[pl.ds(0, SB)], dstbuf.at[0],
                              sem).wait()

    pltpu.sync_copy(zeros1.at[pl.ds(s * ROWS_PER_TILE, ROWS_PER_TILE)],
                    acc.at[pl.ds(s * ROWS_PER_TILE, ROWS_PER_TILE)])
    pltpu.sync_copy(ones1, onesbuf)
    plsc.subcore_barrier()

    base = w * EPW
    pltpu.sync_copy(dst1d.at[pl.ds(base, SB)], dstbuf.at[0])
    pltpu.sync_copy(dst1d.at[pl.ds(base + SB, SB)], dstbuf.at[1])

    @pl.loop(0, SBPW // 4)
    def _quad(t):
        for j in range(4):
            k = j % 2
            q = j

            def _step(t=t, j=j, k=k, q=q):
                sb = t * 4 + j
                drain_2kb(sem_s[k])
                if j < 2:
                    @pl.when(t >= 1)
                    def _di():
                        drain_2kb(sem_i[q])
                else:
                    drain_2kb(sem_i[q])
                pltpu.async_copy(onesbuf, acc.at[dstbuf.at[q]], sem_s[k],
                                 add=True)
                if j < 2:
                    pltpu.async_copy(dst1d.at[pl.ds(base + (sb + 2) * SB,
                                                    SB)],
                                     dstbuf.at[(q + 2) % 4],
                                     sem_i[(q + 2) % 4])
                else:
                    @pl.when(t < SBPW // 4 - 1)
                    def _pf():
                        pltpu.async_copy(dst1d.at[pl.ds(base + (sb + 2) * SB,
                                                        SB)],
                                         dstbuf.at[(q + 2) % 4],
                                         sem_i[(q + 2) % 4])

            if j < 2:
                @pl.when(t >= 1)
                def _ss():
                    _step()

                @pl.when(t < 1)
                def _sp(j=j, k=k, q=q):
                    pltpu.async_copy(onesbuf, acc.at[dstbuf.at[q]],
                                     sem_s[k], add=True)
                    pltpu.async_copy(dst1d.at[pl.ds(base + (j + 2) * SB,
                                                    SB)],
                                     dstbuf.at[q + 2], sem_i[q + 2])
            else:
                _step()

    for k in range(2):
        drain_2kb(sem_s[k])

    plsc.subcore_barrier()
    pltpu.sync_copy(acc.at[pl.ds(s * ROWS_PER_TILE, ROWS_PER_TILE)], degv)
    for chunk in range(ROWS_PER_TILE // DEG_CHUNK):
        @pl.loop(0, DEG_CHUNK // 16)
        def _bcast(i):
            v = degv[pl.ds(chunk * DEG_CHUNK + i * 16, 16)]
            for j in range(16):
                bc[pl.ds((i * 16 + j) * F_PAD, F_PAD)] = (
                    jnp.broadcast_to(v[j], (F_PAD,)))

        pltpu.sync_copy(
            bc,
            out.at[pl.ds((c * NP + s * ROWS_PER_TILE + chunk * DEG_CHUNK)
                         * F_PAD, DEG_CHUNK * F_PAD)])


def _make_agg_kernel(G):
    @functools.partial(
        pl.kernel,
        out_type=jax.ShapeDtypeStruct((2 * G * NP, F_PAD), jnp.float32),
        mesh=_sc_mesh(),
        scratch_types=[
            pltpu.VMEM((4, SB), jnp.int32),
            pltpu.VMEM((4, SB), jnp.int32),
            pltpu.VMEM((2, SB, F_PAD), jnp.float32),
            [pltpu.SemaphoreType.DMA] * 2,
            [pltpu.SemaphoreType.DMA] * 2,
            [pltpu.SemaphoreType.DMA] * 4,
            pltpu.VMEM_SHARED((NP, F_PAD), jnp.float32),
        ],
        compiler_params=_SC_PARAMS,
    )
    def _agg(src1d, dst1d, zeros2, *rest):
        srcs = rest[:G]
        out = rest[G]
        srcbuf, dstbuf, rows, sem_g, sem_s, sem_i, acc = rest[G + 1:]
        c = lax.axis_index("c")
        s = lax.axis_index("s")
        w = c * NS + s

        def load_idx(e0, q, sem=None):
            if sem is None:
                pltpu.sync_copy(src1d.at[pl.ds(e0, SB)], srcbuf.at[q])
                pltpu.sync_copy(dst1d.at[pl.ds(e0, SB)], dstbuf.at[q])
            else:
                pltpu.async_copy(src1d.at[pl.ds(e0, SB)], srcbuf.at[q], sem)
                pltpu.async_copy(dst1d.at[pl.ds(e0, SB)], dstbuf.at[q], sem)

        def drain_idx(q):
            pltpu.make_async_copy(src1d.at[pl.ds(0, SB)], srcbuf.at[q],
                                  sem_i[q]).wait()
            pltpu.make_async_copy(dst1d.at[pl.ds(0, SB)], dstbuf.at[q],
                                  sem_i[q]).wait()

        def start_scatter(o, q):
            pltpu.async_copy(rows.at[o], acc.at[dstbuf.at[q]], sem_s[o],
                             add=True)

        def drain_rows(g, sem):
            pltpu.make_async_copy(srcs[g].at[pl.ds(0, SB)], rows.at[0],
                                  sem).wait()

        for g in range(G):
            pltpu.sync_copy(
                zeros2.at[pl.ds(s * ROWS_PER_TILE, ROWS_PER_TILE)],
                acc.at[pl.ds(s * ROWS_PER_TILE, ROWS_PER_TILE)])
            plsc.subcore_barrier()

            base = w * EPW
            load_idx(base, 0)
            load_idx(base + SB, 1)

            @pl.loop(0, SBPW // 4)
            def _quad(t):
                for j in range(4):
                    k = j % 2
                    o = 1 - k
                    q = j

                    def _step(t=t, j=j, k=k, o=o, q=q):
                        sb = t * 4 + j
                        drain_rows(g, sem_s[k])
                        if j < 2:
                            @pl.when(t >= 1)
                            def _di():
                                drain_idx(q)
                        else:
                            drain_idx(q)
                        pltpu.async_copy(srcs[g].at[srcbuf.at[q]],
                                         rows.at[k], sem_g[k])
                        if j < 2:
                            load_idx(w * EPW + (sb + 2) * SB, (q + 2) % 4,
                                     sem_i[(q + 2) % 4])
                        else:
                            @pl.when(t < SBPW // 4 - 1)
                            def _pf():
                                load_idx(w * EPW + (sb + 2) * SB,
                                         (q + 2) % 4, sem_i[(q + 2) % 4])
                        drain_rows(g, sem_g[o])
                        start_scatter(o, (q + 3) % 4)

                    if j == 0:
                        @pl.when(t >= 1)
                        def _s0():
                            _step()

                        @pl.when(t < 1)
                        def _s0p():
                            pltpu.async_copy(srcs[g].at[srcbuf.at[0]],
                                             rows.at[0], sem_g[0])
                            load_idx(w * EPW + 2 * SB, 2, sem_i[2])
                    elif j == 1:
                        @pl.when(t >= 1)
                        def _s1():
                            _step()

                        @pl.when(t < 1)
                        def _s1p():
                            pltpu.async_copy(srcs[g].at[srcbuf.at[1]],
                                             rows.at[1], sem_g[1])
                            load_idx(w * EPW + 3 * SB, 3, sem_i[3])
                            drain_rows(g, sem_g[0])
                            start_scatter(0, 0)
                    else:
                        _step()

            drain_rows(g, sem_g[1])
            start_scatter(1, 3)
            drain_rows(g, sem_s[0])
            drain_rows(g, sem_s[1])

            plsc.subcore_barrier()
            pltpu.sync_copy(
                acc.at[pl.ds(s * ROWS_PER_TILE, ROWS_PER_TILE)],
                out.at[pl.ds((c * G + g) * NP + s * ROWS_PER_TILE,
                             ROWS_PER_TILE)])
            plsc.subcore_barrier()

    return _agg


_agg1_kernel = _make_agg_kernel(1)
_agg4_kernel = _make_agg_kernel(4)


R = NP // 8
BB = 736
GRID = R // BB


def _stage1_body(deg_ref, xpad_ref, xs_ref, dinv_ref):
    d = deg_ref[0] + deg_ref[1] + 1.0
    dv = lax.rsqrt(d)
    dinv_ref[...] = dv
    xs_ref[...] = xpad_ref[...] * dv


def _stage1(deg16p, xpadp):
    return pl.pallas_call(
        _stage1_body,
        grid=(GRID,),
        in_specs=[
            pl.BlockSpec((2, BB, 128), lambda i: (0, i, 0)),
            pl.BlockSpec((BB, 128), lambda i: (i, 0)),
        ],
        out_specs=[
            pl.BlockSpec((BB, 128), lambda i: (i, 0)),
            pl.BlockSpec((BB, 128), lambda i: (i, 0)),
        ],
        out_shape=[
            jax.ShapeDtypeStruct((R, 128), jnp.float32),
            jax.ShapeDtypeStruct((R, 128), jnp.float32),
        ],
    )(deg16p, xpadp)


def _stage2_body(a1_ref, xs_ref, dinv_ref, w1s_ref, b1c_ref, *out_refs):
    t = (a1_ref[0] + a1_ref[1] + xs_ref[...]) * dinv_ref[...]
    h = jnp.dot(t, w1s_ref[...], preferred_element_type=jnp.float32)
    h = h + b1c_ref[...]
    r = jnp.maximum(h, 0.0)
    dv = dinv_ref[...]
    for g in range(4):
        out_refs[g][...] = r[:, g * 128:(g + 1) * 128] * dv


def _stage2(a1p, xsp, dinvp, w1s, b1c):
    return pl.pallas_call(
        _stage2_body,
        grid=(GRID,),
        in_specs=[
            pl.BlockSpec((2, BB, 128), lambda i: (0, i, 0)),
            pl.BlockSpec((BB, 128), lambda i: (i, 0)),
            pl.BlockSpec((BB, 128), lambda i: (i, 0)),
            pl.BlockSpec((128, 512), lambda i: (0, 0)),
            pl.BlockSpec((1, 512), lambda i: (0, 0)),
        ],
        out_specs=[pl.BlockSpec((BB, 128), lambda i: (i, 0))] * 4,
        out_shape=[jax.ShapeDtypeStruct((R, 128), jnp.float32)] * 4,
    )(a1p, xsp, dinvp, w1s, b1c)


def _stage3_body(a2_ref, h0_ref, h1_ref, h2_ref, h3_ref, dinv_ref,
                 w2s_ref, b2c_ref, wos_ref, boc_ref, sumg_ref, out_ref):
    hs = (h0_ref[...], h1_ref[...], h2_ref[...], h3_ref[...])
    dv = dinv_ref[...]
    t2 = jnp.concatenate(
        [(a2_ref[g] + a2_ref[4 + g] + hs[g]) * dv for g in range(4)], axis=1)
    h2 = jnp.dot(t2, w2s_ref[...], preferred_element_type=jnp.float32)
    h2 = h2 + b2c_ref[...]
    r2 = jnp.maximum(h2, 0.0)
    lg = jnp.dot(r2, wos_ref[...], preferred_element_type=jnp.float32)
    lg = lg + boc_ref[...]
    m = jnp.max(lg, axis=1, keepdims=True)
    e = jnp.exp(lg - m)
    ssum = jnp.dot(e, sumg_ref[...], preferred_element_type=jnp.float32)
    out_ref[...] = e / ssum


def _stage3(a2p, hs4, dinvp, w2s, b2c, wos, boc, sumg):
    return pl.pallas_call(
        _stage3_body,
        grid=(GRID,),
        in_specs=[
            pl.BlockSpec((8, BB, 128), lambda i: (0, i, 0)),
            pl.BlockSpec((BB, 128), lambda i: (i, 0)),
            pl.BlockSpec((BB, 128), lambda i: (i, 0)),
            pl.BlockSpec((BB, 128), lambda i: (i, 0)),
            pl.BlockSpec((BB, 128), lambda i: (i, 0)),
            pl.BlockSpec((BB, 128), lambda i: (i, 0)),
            pl.BlockSpec((512, 512), lambda i: (0, 0)),
            pl.BlockSpec((1, 512), lambda i: (0, 0)),
            pl.BlockSpec((512, 64), lambda i: (0, 0)),
            pl.BlockSpec((1, 64), lambda i: (0, 0)),
            pl.BlockSpec((64, 64), lambda i: (0, 0)),
        ],
        out_specs=pl.BlockSpec((BB, 64), lambda i: (i, 0)),
        out_shape=jax.ShapeDtypeStruct((R, 64), jnp.float32),
    )(a2p, *hs4, dinvp, w2s, b2c, wos, boc, sumg)


def kernel(x, edge_index, batch, W1, b1, W2, b2, Wo, bo):
    del batch
    f_in = x.shape[2]
    eye8 = jnp.eye(8, dtype=jnp.float32)

    x_last = x[:, -1, :]
    xpad = jnp.zeros((NP, F_PAD), jnp.float32).at[:N, :f_in].set(x_last)
    xpadp = xpad.reshape(R, 128)

    pad_idx = (N + (jnp.arange(EP - E, dtype=jnp.int32) % (NP - N)))
    src = jnp.concatenate([edge_index[0], pad_idx])
    dst = jnp.concatenate([edge_index[1], pad_idx])

    zeros1 = jnp.zeros((NP,), jnp.float32)
    zeros2 = jnp.zeros((NP, F_PAD), jnp.float32)
    ones1 = jnp.ones((SB,), jnp.float32)

    w1p = jnp.zeros((F_PAD, H), jnp.float32).at[:f_in, :].set(W1)
    w1s = jnp.concatenate(
        [jnp.kron(eye8, w1p[:, g * 16:(g + 1) * 16]) for g in range(4)],
        axis=1)
    b1c = jnp.tile(b1.reshape(4, 16), (1, 8)).reshape(1, 512)
    w2r = W2.reshape(4, 16, 4, 16)
    w2s = jnp.concatenate(
        [jnp.concatenate([jnp.kron(eye8, w2r[gi, :, go, :])
                          for gi in range(4)], axis=0)
         for go in range(4)], axis=1)
    b2c = jnp.tile(b2.reshape(4, 16), (1, 8)).reshape(1, 512)
    wop = jnp.zeros((H, 8), jnp.float32).at[:, :OUT].set(Wo)
    wos = jnp.concatenate(
        [jnp.kron(eye8, wop[g * 16:(g + 1) * 16, :]) for g in range(4)],
        axis=0)
    bo8 = jnp.full((8,), -1e30, jnp.float32).at[:OUT].set(bo)
    boc = jnp.tile(bo8, 8).reshape(1, 64)
    sumg = jnp.kron(eye8, jnp.ones((8, 8), jnp.float32))

    deg16p = _deg_kernel(dst, zeros1, ones1).reshape(2, R, 128)

    xsp, dinvp = _stage1(deg16p, xpadp)

    a1 = _agg1_kernel(src, dst, zeros2, xsp.reshape(NP, F_PAD))
    a1p = a1.reshape(2, R, 128)

    hs4 = _stage2(a1p, xsp, dinvp, w1s, b1c)

    a2 = _agg4_kernel(src, dst, zeros2,
                      *[h.reshape(NP, F_PAD) for h in hs4])
    a2p = a2.reshape(8, R, 128)

    probs = _stage3(a2p, hs4, dinvp, w2s, b2c, wos, boc, sumg)

    return probs.reshape(NP, 8)[:N, :OUT]
